# Initial kernel scaffold; baseline (speedup 1.0000x reference)
import jax
import jax.numpy as jnp
from jax.experimental import pallas as pl


def kernel(x, edge_index, edge_weight, categories_value, params):
    raise NotImplementedError("write your pallas kernel here")


# speedup vs baseline: 6.4914x; 6.4914x over previous
import functools

import jax
import jax.numpy as jnp
from jax import lax
from jax.experimental import pallas as pl
from jax.experimental.pallas import tpu as pltpu
from jax.experimental.pallas import tpu_sc as plsc

N = 50000
NPAD = 50176
E = 800000
EPAD = 802816
DP = 80
DWA = 48
DWB = 32
DREAL = 72
HALF = NPAD // 2
ACC_ROWS = HALF + 16
RT = NPAD // 32
RH = HALF // 16
ET32 = EPAD // 32
ECH = 3136
ET16 = EPAD // 16
HCH = 512
GBLK = NPAD // 256


def _mesh():
    return plsc.VectorSubcoreMesh(core_axis_name="c", subcore_axis_name="s",
                                  num_cores=2, num_subcores=16)


def _zero16():
    return jnp.zeros((16,), jnp.float32)


def _sc_gather(cats, id_table, emb0, emb1, emb2):
    f32 = jnp.float32
    out_t = [jax.ShapeDtypeStruct((NPAD, 16), f32) for _ in range(4)]

    @functools.partial(
        pl.kernel,
        out_type=out_t,
        mesh=_mesh(),
        compiler_params=pltpu.CompilerParams(use_tc_tiling_on_sc=False, needs_layout_passes=False),
        scratch_types=[
            pltpu.VMEM((RT,), jnp.int32),
            pltpu.VMEM((RT, 16), f32),
            pltpu.SemaphoreType.DMA,
        ],
    )
    def body(cats_h, t0, t1, t2, t3, o0, o1, o2, o3, idx_v, rows_v, sem):
        c = lax.axis_index("c")
        s = lax.axis_index("s")
        base = (s * 2 + c) * RT
        for k, tbl, out in ((0, t0, o0), (1, t1, o1), (2, t2, o2), (3, t3, o3)):
            pltpu.sync_copy(cats_h.at[pl.ds(k * NPAD + base, RT)], idx_v)
            pltpu.async_copy(tbl.at[idx_v], rows_v, sem).wait()
            pltpu.sync_copy(rows_v, out.at[pl.ds(base, RT)])

    return body(cats, id_table, emb0, emb1, emb2)


def _sc_degree(dst, ew):
    f32 = jnp.float32

    @functools.partial(
        pl.kernel,
        out_type=jax.ShapeDtypeStruct((2 * NPAD,), f32),
        mesh=_mesh(),
        compiler_params=pltpu.CompilerParams(use_tc_tiling_on_sc=False, needs_layout_passes=False),
        scratch_types=[
            pltpu.VMEM((ECH,), jnp.int32),
            pltpu.VMEM((ECH,), f32),
            pltpu.VMEM((ECH,), f32),
            pltpu.VMEM_SHARED((NPAD,), f32),
        ],
    )
    def body(dst_h, ew_h, out, idx_v, val_v, z_v, acc):
        c = lax.axis_index("c")
        s = lax.axis_index("s")

        def zb(i, _):
            z_v[pl.ds(i * 16, 16)] = _zero16()
            return 0

        lax.fori_loop(0, ECH // 16, zb, 0)
        pltpu.sync_copy(z_v, acc.at[pl.ds(s * ECH, ECH)])
        plsc.subcore_barrier()

        def chunk(j, _):
            base = c * (EPAD // 2) + s * ET32 + j * ECH
            pltpu.sync_copy(dst_h.at[pl.ds(base, ECH)], idx_v)
            pltpu.sync_copy(ew_h.at[pl.ds(base, ECH)], val_v)
            pltpu.sync_copy(val_v, acc.at[idx_v], add=True)
            return 0

        lax.fori_loop(0, ET32 // ECH, chunk, 0)
        plsc.subcore_barrier()
        pltpu.sync_copy(acc.at[pl.ds(s * ECH, ECH)], z_v)
        pltpu.sync_copy(z_v, out.at[pl.ds(c * NPAD + s * ECH, ECH)])

    return body(dst, ew)


def _sc_norm(dinv, src, dst, ew):
    f32 = jnp.float32
    i32 = jnp.int32
    out_t = [
        jax.ShapeDtypeStruct((EPAD,), f32),
        jax.ShapeDtypeStruct((2 * EPAD,), i32),
    ]

    @functools.partial(
        pl.kernel,
        out_type=out_t,
        mesh=_mesh(),
        compiler_params=pltpu.CompilerParams(use_tc_tiling_on_sc=False, needs_layout_passes=False),
        scratch_types=[
            pltpu.VMEM((NPAD,), f32),
            pltpu.VMEM((ECH,), i32),
            pltpu.VMEM((ECH,), i32),
            pltpu.VMEM((ECH,), f32),
            pltpu.VMEM((ECH,), f32),
            pltpu.VMEM((ECH,), i32),
            pltpu.VMEM((ECH,), i32),
        ],
    )
    def body(dinv_h, src_h, dst_h, ew_h, norm_o, dstl_o,
             dinv_v, src_v, dst_v, ew_v, nrm_v, d0_v, d1_v):
        c = lax.axis_index("c")
        s = lax.axis_index("s")
        wid = s * 2 + c
        dummy = HALF + lax.rem(wid, 16)
        pltpu.sync_copy(dinv_h, dinv_v)

        def chunk(j, _):
            base = wid * ET32 + j * ECH
            pltpu.sync_copy(src_h.at[pl.ds(base, ECH)], src_v)
            pltpu.sync_copy(dst_h.at[pl.ds(base, ECH)], dst_v)
            pltpu.sync_copy(ew_h.at[pl.ds(base, ECH)], ew_v)

            def grp(g, _):
                sl = pl.ds(g * 16, 16)
                si = src_v[sl]
                di = dst_v[sl]
                a = plsc.load_gather(dinv_v, [si])
                b = plsc.load_gather(dinv_v, [di])
                nrm_v[sl] = a * ew_v[sl] * b
                inhalf = di < HALF
                d0_v[sl] = jnp.where(inhalf, di, dummy)
                d1_v[sl] = jnp.where(inhalf, dummy, di - HALF)
                return 0

            lax.fori_loop(0, ECH // 16, grp, 0)
            pltpu.sync_copy(nrm_v, norm_o.at[pl.ds(base, ECH)])
            pltpu.sync_copy(d0_v, dstl_o.at[pl.ds(base, ECH)])
            pltpu.sync_copy(d1_v, dstl_o.at[pl.ds(EPAD + base, ECH)])
            return 0

        lax.fori_loop(0, ET32 // ECH, chunk, 0)

    return body(dinv, src, dst, ew)


def _sc_hop_half(cur, src, norm, dstl, dw):
    f32 = jnp.float32
    i32 = jnp.int32

    @functools.partial(
        pl.kernel,
        out_type=jax.ShapeDtypeStruct((NPAD, dw), f32),
        mesh=_mesh(),
        compiler_params=pltpu.CompilerParams(use_tc_tiling_on_sc=False, needs_layout_passes=False),
        scratch_types=[
            pltpu.VMEM((HCH,), i32),
            pltpu.VMEM((HCH,), f32),
            pltpu.VMEM((HCH,), i32),
            pltpu.VMEM((HCH, dw), f32),
            pltpu.VMEM((392, dw), f32),
            pltpu.VMEM_SHARED((ACC_ROWS, dw), f32),
            pltpu.SemaphoreType.DMA,
        ],
    )
    def body(cur_h, src_h, norm_h, dstl_h, out,
             src_v, nrm_v, dl_v, rows_v, zb_v, acc, sem):
        c = lax.axis_index("c")
        s = lax.axis_index("s")

        def zb(i, _):
            for j in range(dw // 16):
                zb_v[i, pl.ds(j * 16, 16)] = _zero16()
            return 0

        lax.fori_loop(0, 392, zb, 0)
        for q in range(4):
            pltpu.sync_copy(zb_v, acc.at[pl.ds(s * RH + q * 392, 392)])

        @pl.when(s == 0)
        def _():
            pltpu.sync_copy(zb_v.at[pl.ds(0, 16)], acc.at[pl.ds(HALF, 16)])

        plsc.subcore_barrier()

        def chunk(jc, _):
            base = s * ET16 + jc * HCH
            pltpu.sync_copy(src_h.at[pl.ds(base, HCH)], src_v)
            pltpu.sync_copy(norm_h.at[pl.ds(base, HCH)], nrm_v)
            pltpu.sync_copy(dstl_h.at[pl.ds(c * EPAD + base, HCH)], dl_v)
            pltpu.async_copy(cur_h.at[src_v], rows_v, sem).wait()

            def grp(g, _):
                nv = nrm_v[pl.ds(g * 16, 16)]
                for i in range(16):
                    e = g * 16 + i
                    b = nv.at[jnp.full((16,), i, i32)].get(
                        mode="promise_in_bounds")
                    for j in range(dw // 16):
                        sl = pl.ds(j * 16, 16)
                        rows_v[e, sl] = rows_v[e, sl] * b
                return 0

            lax.fori_loop(0, HCH // 16, grp, 0)
            pltpu.sync_copy(rows_v, acc.at[dl_v], add=True)
            return 0

        lax.fori_loop(0, ET16 // HCH, chunk, 0)
        plsc.subcore_barrier()
        for q in range(4):
            pltpu.sync_copy(acc.at[pl.ds(s * RH + q * 392, 392)], zb_v)
            pltpu.sync_copy(zb_v, out.at[pl.ds(c * HALF + s * RH + q * 392,
                                               392)])

    return body(cur, src, norm, dstl)


def _sc_hop(curA, curB, src, norm, dstl):
    return (_sc_hop_half(curA, src, norm, dstl, DWA),
            _sc_hop_half(curB, src, norm, dstl, DWB))


def _elu(v):
    return jnp.where(v > 0, v, jnp.exp(jnp.minimum(v, 0.0)) - 1.0)


def _full2d(a):
    return pl.BlockSpec(a.shape, lambda i: (0,) * a.ndim)


def _tc_stage1(xp, idr, e0, e1, e2, d0, d1, p):
    f32 = jnp.float32

    def body(x_r, id_r, e0_r, e1_r, e2_r, d0_r, d1_r,
             wid_r, bid_r, w0_r, b0_r, we_r, be_r, g0_r, gb_r,
             ha_o, hb_o, dinv_o):
        idv = _elu(jnp.dot(id_r[...], wid_r[...],
                           preferred_element_type=f32) + bid_r[...])
        h0 = _elu(jnp.dot(x_r[...], w0_r[...],
                          preferred_element_type=f32) + b0_r[...])
        ecat = jnp.concatenate(
            [e0_r[...][:, :8], e1_r[...][:, :8], e2_r[...][:, :8]], axis=1)
        ev = _elu(jnp.dot(ecat, we_r[...],
                          preferred_element_type=f32) + be_r[...])
        hcat = jnp.concatenate([idv, h0, ev], axis=1)
        mu = jnp.mean(hcat, axis=1, keepdims=True)
        var = jnp.mean((hcat - mu) * (hcat - mu), axis=1, keepdims=True)
        hn = (hcat - mu) * lax.rsqrt(var + 1e-5) * g0_r[...] + gb_r[...]
        hp = jnp.concatenate([hn, jnp.zeros((256, DP - DREAL), f32)], axis=1)
        ha_o[...] = hp[:, :DWA]
        hb_o[...] = hp[:, DWA:]
        deg = d0_r[...] + d1_r[...]
        dinv_o[...] = jnp.where(deg > 0, lax.rsqrt(jnp.maximum(deg, 1e-30)),
                                0.0)

    wid = p['W_id']; bid = p['b_id'].reshape(1, -1)
    w0 = p['W0']; b0 = p['b0'].reshape(1, -1)
    we = p['W_emb']; be = p['b_emb'].reshape(1, -1)
    g0 = p['ln0_g'].reshape(1, -1); gb = p['ln0_b'].reshape(1, -1)
    row = lambda shp: pl.BlockSpec(shp, lambda i: (i, 0))
    row3 = pl.BlockSpec((1, 1, 256), lambda i: (i, 0, 0))
    return pl.pallas_call(
        body,
        grid=(GBLK,),
        in_specs=[row((256, 16)), row((256, 16)), row((256, 16)),
                  row((256, 16)), row((256, 16)), row3, row3,
                  _full2d(wid), _full2d(bid), _full2d(w0), _full2d(b0),
                  _full2d(we), _full2d(be), _full2d(g0), _full2d(gb)],
        out_specs=[row((256, DWA)), row((256, DWB)), row3],
        out_shape=[jax.ShapeDtypeStruct((NPAD, DWA), f32),
                   jax.ShapeDtypeStruct((NPAD, DWB), f32),
                   jax.ShapeDtypeStruct((GBLK, 1, 256), f32)],
    )(xp, idr, e0, e1, e2, d0, d1, wid, bid, w0, b0, we, be, g0, gb)


def _tc_final(halves, wall, p):
    f32 = jnp.float32

    def body(*refs):
        (ha_r, hb_r, c1a_r, c1b_r, c2a_r, c2b_r, c3a_r, c3b_r,
         wall_r, tb_r, g1_r, gb_r, w1_r, b1_r, o) = refs
        wr = wall_r[...]
        refs = (ha_r, hb_r, c1a_r, c1b_r, c2a_r, c2b_r, c3a_r, c3b_r)
        off = 0
        out = None
        for q, rr in enumerate(refs):
            w = DWA if q % 2 == 0 else DWB
            term = jnp.dot(rr[...], wr[off:off + w, :],
                           preferred_element_type=f32)
            out = term if out is None else out + term
            off += w
        out = jnp.maximum(out + tb_r[...], 0.0)
        mu = jnp.mean(out, axis=1, keepdims=True)
        var = jnp.mean((out - mu) * (out - mu), axis=1, keepdims=True)
        out = (out - mu) * lax.rsqrt(var + 1e-5) * g1_r[...] + gb_r[...]
        y = jnp.dot(out, w1_r[...], preferred_element_type=f32) + b1_r[...]
        m = jnp.max(y, axis=1, keepdims=True)
        z = y - m
        o[...] = z - jnp.log(jnp.sum(jnp.exp(z), axis=1, keepdims=True))

    tb = p['tag_b'].reshape(1, -1)
    g1 = p['ln1_g'].reshape(1, -1); gb = p['ln1_b'].reshape(1, -1)
    w1 = p['W1']; b1 = p['b1'].reshape(1, -1)
    row = lambda shp: pl.BlockSpec(shp, lambda i: (i, 0))
    return pl.pallas_call(
        body,
        grid=(GBLK,),
        in_specs=[row((256, DWA)), row((256, DWB))] * 4 + [_full2d(wall), _full2d(tb),
                  _full2d(g1), _full2d(gb), _full2d(w1), _full2d(b1)],
        out_specs=row((256, 2)),
        out_shape=jax.ShapeDtypeStruct((NPAD, 2), f32),
    )(*halves, wall, tb, g1, gb, w1, b1)


def kernel(x, edge_index, edge_weight, categories_value, params):
    f32 = jnp.float32
    p = params
    xp = jnp.pad(x, ((0, NPAD - N), (0, 0)))
    src = jnp.pad(edge_index[0], (0, EPAD - E))
    dst = jnp.pad(edge_index[1], (0, EPAD - E))
    ew = jnp.pad(edge_weight, (0, EPAD - E))
    cats = jnp.pad(categories_value.T.astype(jnp.int32),
                   ((0, 0), (0, NPAD - N))).reshape(4 * NPAD)
    embp = jnp.pad(p['emb_tables'], ((0, 0), (0, 0), (0, 8)))
    wall = jnp.pad(p['tag_W'], ((0, 0), (0, DP - DREAL), (0, 0)))
    wall = wall.reshape(4 * DP, -1)

    idr, e0r, e1r, e2r = _sc_gather(cats, p['id_table'], embp[0], embp[1],
                                    embp[2])
    deg2 = _sc_degree(dst, ew).reshape(2, NPAD)
    ha, hb, dinv3 = _tc_stage1(xp, idr, e0r, e1r, e2r,
                               deg2[0].reshape(GBLK, 1, 256),
                               deg2[1].reshape(GBLK, 1, 256), p)
    dinv = dinv3.reshape(NPAD)
    norm, dstl = _sc_norm(dinv, src, dst, ew)
    c1a, c1b = _sc_hop(ha, hb, src, norm, dstl)
    c2a, c2b = _sc_hop(c1a, c1b, src, norm, dstl)
    c3a, c3b = _sc_hop(c2a, c2b, src, norm, dstl)
    out = _tc_final((ha, hb, c1a, c1b, c2a, c2b, c3a, c3b), wall, p)
    return out[:N]


# speedup vs baseline: 7.7965x; 1.2010x over previous
import functools

import jax
import jax.numpy as jnp
from jax import lax
from jax.experimental import pallas as pl
from jax.experimental.pallas import tpu as pltpu
from jax.experimental.pallas import tpu_sc as plsc

N = 50000
NPAD = 50176
E = 800000
EPAD = 802816
DP = 80
DWA = 48
DWB = 32
DREAL = 72
HALF = NPAD // 2
ACC_ROWS = HALF + 16
RT = NPAD // 32
RH = HALF // 16
ET32 = EPAD // 32
ECH = 3136
ET16 = EPAD // 16
HCH = 448
GBLK = NPAD // 256


def _mesh():
    return plsc.VectorSubcoreMesh(core_axis_name="c", subcore_axis_name="s",
                                  num_cores=2, num_subcores=16)


def _zero16():
    return jnp.zeros((16,), jnp.float32)


def _sc_gather(cats, id_table, emb0, emb1, emb2):
    f32 = jnp.float32
    out_t = [jax.ShapeDtypeStruct((NPAD, 16), f32) for _ in range(4)]

    @functools.partial(
        pl.kernel,
        out_type=out_t,
        mesh=_mesh(),
        compiler_params=pltpu.CompilerParams(use_tc_tiling_on_sc=False, needs_layout_passes=False),
        scratch_types=[
            pltpu.VMEM((RT,), jnp.int32),
            pltpu.VMEM((RT, 16), f32),
            pltpu.SemaphoreType.DMA,
        ],
    )
    def body(cats_h, t0, t1, t2, t3, o0, o1, o2, o3, idx_v, rows_v, sem):
        c = lax.axis_index("c")
        s = lax.axis_index("s")
        base = (s * 2 + c) * RT
        for k, tbl, out in ((0, t0, o0), (1, t1, o1), (2, t2, o2), (3, t3, o3)):
            pltpu.sync_copy(cats_h.at[pl.ds(k * NPAD + base, RT)], idx_v)
            pltpu.async_copy(tbl.at[idx_v], rows_v, sem).wait()
            pltpu.sync_copy(rows_v, out.at[pl.ds(base, RT)])

    return body(cats, id_table, emb0, emb1, emb2)


def _sc_degree(dst, ew):
    f32 = jnp.float32

    @functools.partial(
        pl.kernel,
        out_type=jax.ShapeDtypeStruct((2 * NPAD,), f32),
        mesh=_mesh(),
        compiler_params=pltpu.CompilerParams(use_tc_tiling_on_sc=False, needs_layout_passes=False),
        scratch_types=[
            pltpu.VMEM((ECH,), jnp.int32),
            pltpu.VMEM((ECH,), f32),
            pltpu.VMEM((ECH,), f32),
            pltpu.VMEM_SHARED((NPAD,), f32),
        ],
    )
    def body(dst_h, ew_h, out, idx_v, val_v, z_v, acc):
        c = lax.axis_index("c")
        s = lax.axis_index("s")

        def zb(i, _):
            z_v[pl.ds(i * 16, 16)] = _zero16()
            return 0

        lax.fori_loop(0, ECH // 16, zb, 0)
        pltpu.sync_copy(z_v, acc.at[pl.ds(s * ECH, ECH)])
        plsc.subcore_barrier()

        def chunk(j, _):
            base = c * (EPAD // 2) + s * ET32 + j * ECH
            pltpu.sync_copy(dst_h.at[pl.ds(base, ECH)], idx_v)
            pltpu.sync_copy(ew_h.at[pl.ds(base, ECH)], val_v)
            pltpu.sync_copy(val_v, acc.at[idx_v], add=True)
            return 0

        lax.fori_loop(0, ET32 // ECH, chunk, 0)
        plsc.subcore_barrier()
        pltpu.sync_copy(acc.at[pl.ds(s * ECH, ECH)], z_v)
        pltpu.sync_copy(z_v, out.at[pl.ds(c * NPAD + s * ECH, ECH)])

    return body(dst, ew)


def _sc_norm(dinv, src, dst, ew):
    f32 = jnp.float32
    i32 = jnp.int32
    out_t = [
        jax.ShapeDtypeStruct((EPAD,), f32),
        jax.ShapeDtypeStruct((2 * EPAD,), i32),
    ]

    @functools.partial(
        pl.kernel,
        out_type=out_t,
        mesh=_mesh(),
        compiler_params=pltpu.CompilerParams(use_tc_tiling_on_sc=False, needs_layout_passes=False),
        scratch_types=[
            pltpu.VMEM((NPAD,), f32),
            pltpu.VMEM((ECH,), i32),
            pltpu.VMEM((ECH,), i32),
            pltpu.VMEM((ECH,), f32),
            pltpu.VMEM((ECH,), f32),
            pltpu.VMEM((ECH,), i32),
            pltpu.VMEM((ECH,), i32),
        ],
    )
    def body(dinv_h, src_h, dst_h, ew_h, norm_o, dstl_o,
             dinv_v, src_v, dst_v, ew_v, nrm_v, d0_v, d1_v):
        c = lax.axis_index("c")
        s = lax.axis_index("s")
        wid = s * 2 + c
        dummy = HALF + lax.rem(wid, 16)
        pltpu.sync_copy(dinv_h, dinv_v)

        def chunk(j, _):
            base = wid * ET32 + j * ECH
            pltpu.sync_copy(src_h.at[pl.ds(base, ECH)], src_v)
            pltpu.sync_copy(dst_h.at[pl.ds(base, ECH)], dst_v)
            pltpu.sync_copy(ew_h.at[pl.ds(base, ECH)], ew_v)

            def grp(g, _):
                sl = pl.ds(g * 16, 16)
                si = src_v[sl]
                di = dst_v[sl]
                a = plsc.load_gather(dinv_v, [si])
                b = plsc.load_gather(dinv_v, [di])
                nrm_v[sl] = a * ew_v[sl] * b
                inhalf = di < HALF
                d0_v[sl] = jnp.where(inhalf, di, dummy)
                d1_v[sl] = jnp.where(inhalf, dummy, di - HALF)
                return 0

            lax.fori_loop(0, ECH // 16, grp, 0)
            pltpu.sync_copy(nrm_v, norm_o.at[pl.ds(base, ECH)])
            pltpu.sync_copy(d0_v, dstl_o.at[pl.ds(base, ECH)])
            pltpu.sync_copy(d1_v, dstl_o.at[pl.ds(EPAD + base, ECH)])
            return 0

        lax.fori_loop(0, ET32 // ECH, chunk, 0)

    return body(dinv, src, dst, ew)


def _sc_hop_half(cur, src, norm, dstl, dw):
    f32 = jnp.float32
    i32 = jnp.int32
    NCH = ET16 // HCH

    @functools.partial(
        pl.kernel,
        out_type=jax.ShapeDtypeStruct((NPAD, dw), f32),
        mesh=_mesh(),
        compiler_params=pltpu.CompilerParams(use_tc_tiling_on_sc=False, needs_layout_passes=False),
        scratch_types=[
            pltpu.VMEM((HCH,), i32), pltpu.VMEM((HCH,), f32),
            pltpu.VMEM((HCH,), i32),
            pltpu.VMEM((HCH,), i32), pltpu.VMEM((HCH,), f32),
            pltpu.VMEM((HCH,), i32),
            pltpu.VMEM((HCH, dw), f32), pltpu.VMEM((HCH, dw), f32),
            pltpu.VMEM_SHARED((ACC_ROWS, dw), f32),
            pltpu.SemaphoreType.DMA, pltpu.SemaphoreType.DMA,
        ],
    )
    def body(cur_h, src_h, norm_h, dstl_h, out,
             s0, n0, d0, s1, n1, d1, r0, r1, acc, g0, g1):
        c = lax.axis_index("c")
        s = lax.axis_index("s")

        def zr(i, _):
            for j in range(dw // 16):
                r0[i, pl.ds(j * 16, 16)] = _zero16()
            return 0

        lax.fori_loop(0, HCH, zr, 0)
        for q in range(3):
            pltpu.sync_copy(r0, acc.at[pl.ds(s * RH + q * 448, 448)])
        pltpu.sync_copy(r0.at[pl.ds(0, 224)],
                        acc.at[pl.ds(s * RH + 3 * 448, 224)])

        @pl.when(s == 0)
        def _():
            pltpu.sync_copy(r0.at[pl.ds(0, 16)], acc.at[pl.ds(HALF, 16)])

        plsc.subcore_barrier()
        ebase = s * ET16

        def fetch_idx(j, sv, nv, dv):
            b = ebase + j * HCH
            pltpu.sync_copy(src_h.at[pl.ds(b, HCH)], sv)
            pltpu.sync_copy(norm_h.at[pl.ds(b, HCH)], nv)
            pltpu.sync_copy(dstl_h.at[pl.ds(c * EPAD + b, HCH)], dv)

        def scale(rv, nv):
            def grp(g, _):
                nvv = nv[pl.ds(g * 16, 16)]
                for i in range(16):
                    e = g * 16 + i
                    bb = nvv.at[jnp.full((16,), i, i32)].get(
                        mode="promise_in_bounds")
                    for jj in range(dw // 16):
                        sl = pl.ds(jj * 16, 16)
                        rv[e, sl] = rv[e, sl] * bb
                return 0

            lax.fori_loop(0, HCH // 16, grp, 0)

        fetch_idx(0, s0, n0, d0)
        pltpu.async_copy(cur_h.at[s0], r0, g0)
        fetch_idx(1, s1, n1, d1)

        def it(i, _):
            j = i * 2
            pltpu.make_async_copy(cur_h.at[s0], r0, g0).wait()
            pltpu.async_copy(cur_h.at[s1], r1, g1)
            scale(r0, n0)
            pltpu.sync_copy(r0, acc.at[d0], add=True)

            @pl.when(j + 2 < NCH)
            def _():
                fetch_idx(j + 2, s0, n0, d0)

            pltpu.make_async_copy(cur_h.at[s1], r1, g1).wait()

            @pl.when(j + 2 < NCH)
            def _():
                pltpu.async_copy(cur_h.at[s0], r0, g0)

            scale(r1, n1)
            pltpu.sync_copy(r1, acc.at[d1], add=True)

            @pl.when(j + 3 < NCH)
            def _():
                fetch_idx(j + 3, s1, n1, d1)

            return 0

        lax.fori_loop(0, NCH // 2, it, 0)
        plsc.subcore_barrier()
        for q in range(3):
            pltpu.sync_copy(acc.at[pl.ds(s * RH + q * 448, 448)], r0)
            pltpu.sync_copy(r0, out.at[pl.ds(c * HALF + s * RH + q * 448,
                                             448)])
        pltpu.sync_copy(acc.at[pl.ds(s * RH + 3 * 448, 224)],
                        r0.at[pl.ds(0, 224)])
        pltpu.sync_copy(r0.at[pl.ds(0, 224)],
                        out.at[pl.ds(c * HALF + s * RH + 3 * 448, 224)])

    return body(cur, src, norm, dstl)


def _sc_hop(curA, curB, src, norm, dstl):
    return (_sc_hop_half(curA, src, norm, dstl, DWA),
            _sc_hop_half(curB, src, norm, dstl, DWB))


def _elu(v):
    return jnp.where(v > 0, v, jnp.exp(jnp.minimum(v, 0.0)) - 1.0)


def _full2d(a):
    return pl.BlockSpec(a.shape, lambda i: (0,) * a.ndim)


def _tc_stage1(xp, idr, e0, e1, e2, d0, d1, p):
    f32 = jnp.float32

    def body(x_r, id_r, e0_r, e1_r, e2_r, d0_r, d1_r,
             wid_r, bid_r, w0_r, b0_r, we_r, be_r, g0_r, gb_r,
             ha_o, hb_o, dinv_o):
        idv = _elu(jnp.dot(id_r[...], wid_r[...],
                           preferred_element_type=f32) + bid_r[...])
        h0 = _elu(jnp.dot(x_r[...], w0_r[...],
                          preferred_element_type=f32) + b0_r[...])
        ecat = jnp.concatenate(
            [e0_r[...][:, :8], e1_r[...][:, :8], e2_r[...][:, :8]], axis=1)
        ev = _elu(jnp.dot(ecat, we_r[...],
                          preferred_element_type=f32) + be_r[...])
        hcat = jnp.concatenate([idv, h0, ev], axis=1)
        mu = jnp.mean(hcat, axis=1, keepdims=True)
        var = jnp.mean((hcat - mu) * (hcat - mu), axis=1, keepdims=True)
        hn = (hcat - mu) * lax.rsqrt(var + 1e-5) * g0_r[...] + gb_r[...]
        hp = jnp.concatenate([hn, jnp.zeros((256, DP - DREAL), f32)], axis=1)
        ha_o[...] = hp[:, :DWA]
        hb_o[...] = hp[:, DWA:]
        deg = d0_r[...] + d1_r[...]
        dinv_o[...] = jnp.where(deg > 0, lax.rsqrt(jnp.maximum(deg, 1e-30)),
                                0.0)

    wid = p['W_id']; bid = p['b_id'].reshape(1, -1)
    w0 = p['W0']; b0 = p['b0'].reshape(1, -1)
    we = p['W_emb']; be = p['b_emb'].reshape(1, -1)
    g0 = p['ln0_g'].reshape(1, -1); gb = p['ln0_b'].reshape(1, -1)
    row = lambda shp: pl.BlockSpec(shp, lambda i: (i, 0))
    row3 = pl.BlockSpec((1, 1, 256), lambda i: (i, 0, 0))
    return pl.pallas_call(
        body,
        grid=(GBLK,),
        in_specs=[row((256, 16)), row((256, 16)), row((256, 16)),
                  row((256, 16)), row((256, 16)), row3, row3,
                  _full2d(wid), _full2d(bid), _full2d(w0), _full2d(b0),
                  _full2d(we), _full2d(be), _full2d(g0), _full2d(gb)],
        out_specs=[row((256, DWA)), row((256, DWB)), row3],
        out_shape=[jax.ShapeDtypeStruct((NPAD, DWA), f32),
                   jax.ShapeDtypeStruct((NPAD, DWB), f32),
                   jax.ShapeDtypeStruct((GBLK, 1, 256), f32)],
    )(xp, idr, e0, e1, e2, d0, d1, wid, bid, w0, b0, we, be, g0, gb)


def _tc_final(halves, wall, p):
    f32 = jnp.float32

    def body(*refs):
        (ha_r, hb_r, c1a_r, c1b_r, c2a_r, c2b_r, c3a_r, c3b_r,
         wall_r, tb_r, g1_r, gb_r, w1_r, b1_r, o) = refs
        wr = wall_r[...]
        refs = (ha_r, hb_r, c1a_r, c1b_r, c2a_r, c2b_r, c3a_r, c3b_r)
        off = 0
        out = None
        for q, rr in enumerate(refs):
            w = DWA if q % 2 == 0 else DWB
            term = jnp.dot(rr[...], wr[off:off + w, :],
                           preferred_element_type=f32)
            out = term if out is None else out + term
            off += w
        out = jnp.maximum(out + tb_r[...], 0.0)
        mu = jnp.mean(out, axis=1, keepdims=True)
        var = jnp.mean((out - mu) * (out - mu), axis=1, keepdims=True)
        out = (out - mu) * lax.rsqrt(var + 1e-5) * g1_r[...] + gb_r[...]
        y = jnp.dot(out, w1_r[...], preferred_element_type=f32) + b1_r[...]
        m = jnp.max(y, axis=1, keepdims=True)
        z = y - m
        o[...] = z - jnp.log(jnp.sum(jnp.exp(z), axis=1, keepdims=True))

    tb = p['tag_b'].reshape(1, -1)
    g1 = p['ln1_g'].reshape(1, -1); gb = p['ln1_b'].reshape(1, -1)
    w1 = p['W1']; b1 = p['b1'].reshape(1, -1)
    row = lambda shp: pl.BlockSpec(shp, lambda i: (i, 0))
    return pl.pallas_call(
        body,
        grid=(GBLK,),
        in_specs=[row((256, DWA)), row((256, DWB))] * 4 + [_full2d(wall), _full2d(tb),
                  _full2d(g1), _full2d(gb), _full2d(w1), _full2d(b1)],
        out_specs=row((256, 2)),
        out_shape=jax.ShapeDtypeStruct((NPAD, 2), f32),
    )(*halves, wall, tb, g1, gb, w1, b1)


def kernel(x, edge_index, edge_weight, categories_value, params):
    f32 = jnp.float32
    p = params
    xp = jnp.pad(x, ((0, NPAD - N), (0, 0)))
    src = jnp.pad(edge_index[0], (0, EPAD - E))
    dst = jnp.pad(edge_index[1], (0, EPAD - E))
    ew = jnp.pad(edge_weight, (0, EPAD - E))
    cats = jnp.pad(categories_value.T.astype(jnp.int32),
                   ((0, 0), (0, NPAD - N))).reshape(4 * NPAD)
    embp = jnp.pad(p['emb_tables'], ((0, 0), (0, 0), (0, 8)))
    wall = jnp.pad(p['tag_W'], ((0, 0), (0, DP - DREAL), (0, 0)))
    wall = wall.reshape(4 * DP, -1)

    idr, e0r, e1r, e2r = _sc_gather(cats, p['id_table'], embp[0], embp[1],
                                    embp[2])
    deg2 = _sc_degree(dst, ew).reshape(2, NPAD)
    ha, hb, dinv3 = _tc_stage1(xp, idr, e0r, e1r, e2r,
                               deg2[0].reshape(GBLK, 1, 256),
                               deg2[1].reshape(GBLK, 1, 256), p)
    dinv = dinv3.reshape(NPAD)
    norm, dstl = _sc_norm(dinv, src, dst, ew)
    c1a, c1b = _sc_hop(ha, hb, src, norm, dstl)
    c2a, c2b = _sc_hop(c1a, c1b, src, norm, dstl)
    c3a, c3b = _sc_hop(c2a, c2b, src, norm, dstl)
    out = _tc_final((ha, hb, c1a, c1b, c2a, c2b, c3a, c3b), wall, p)
    return out[:N]


# speedup vs baseline: 12.0539x; 1.5461x over previous
import functools

import jax
import jax.numpy as jnp
from jax import lax
from jax.experimental import pallas as pl
from jax.experimental.pallas import tpu as pltpu
from jax.experimental.pallas import tpu_sc as plsc

N = 50000
NPAD = 50176
E = 800000
EPAD = 802816
DP = 80
DWA = 48
DWB = 32
DREAL = 72
HALF = NPAD // 2
RT = NPAD // 32
RH = HALF // 16
ET32 = EPAD // 32
ECH = 3136
ET16 = EPAD // 16
HCH = 448
GBLK = NPAD // 256
PKSH = EPAD + 256


def _mesh():
    return plsc.VectorSubcoreMesh(core_axis_name="c", subcore_axis_name="s",
                                  num_cores=2, num_subcores=16)


_SCPARAMS = dict(
    compiler_params=pltpu.CompilerParams(use_tc_tiling_on_sc=False,
                                         needs_layout_passes=False))


def _zero16():
    return jnp.zeros((16,), jnp.float32)


def _iota16():
    return lax.iota(jnp.int32, 16)


def _sc_gather(cats, id_table, emb0, emb1, emb2):
    f32 = jnp.float32
    out_t = [jax.ShapeDtypeStruct((NPAD, 16), f32) for _ in range(4)]

    @functools.partial(
        pl.kernel,
        out_type=out_t,
        mesh=_mesh(),
        scratch_types=[
            pltpu.VMEM((RT,), jnp.int32),
            pltpu.VMEM((RT, 16), f32),
            pltpu.SemaphoreType.DMA,
        ],
        **_SCPARAMS,
    )
    def body(cats_h, t0, t1, t2, t3, o0, o1, o2, o3, idx_v, rows_v, sem):
        c = lax.axis_index("c")
        s = lax.axis_index("s")
        base = (s * 2 + c) * RT
        for k, tbl, out in ((0, t0, o0), (1, t1, o1), (2, t2, o2), (3, t3, o3)):
            pltpu.sync_copy(cats_h.at[pl.ds(k * NPAD + base, RT)], idx_v)
            pltpu.async_copy(tbl.at[idx_v], rows_v, sem).wait()
            pltpu.sync_copy(rows_v, out.at[pl.ds(base, RT)])

    return body(cats, id_table, emb0, emb1, emb2)


def _sc_degree(dst, ew):
    f32 = jnp.float32
    i32 = jnp.int32
    out_t = [jax.ShapeDtypeStruct((2 * NPAD,), f32),
             jax.ShapeDtypeStruct((512,), i32)]

    @functools.partial(
        pl.kernel,
        out_type=out_t,
        mesh=_mesh(),
        scratch_types=[
            pltpu.VMEM((ECH,), i32),
            pltpu.VMEM((ECH,), f32),
            pltpu.VMEM((ECH,), f32),
            pltpu.VMEM((16,), i32),
            pltpu.VMEM_SHARED((NPAD,), f32),
        ],
        **_SCPARAMS,
    )
    def body(dst_h, ew_h, out, cnt_o, idx_v, val_v, z_v, cnt_v, acc):
        c = lax.axis_index("c")
        s = lax.axis_index("s")
        wid = s * 2 + c

        def zb(i, _):
            z_v[pl.ds(i * 16, 16)] = _zero16()
            return 0

        lax.fori_loop(0, ECH // 16, zb, 0)
        pltpu.sync_copy(z_v, acc.at[pl.ds(s * ECH, ECH)])
        plsc.subcore_barrier()

        def chunk(j, cnt):
            base = wid * ET32 + j * ECH
            pltpu.sync_copy(dst_h.at[pl.ds(base, ECH)], idx_v)
            pltpu.sync_copy(ew_h.at[pl.ds(base, ECH)], val_v)
            pltpu.sync_copy(val_v, acc.at[idx_v], add=True)

            def grp(g, cn):
                di = idx_v[pl.ds(g * 16, 16)]
                return cn + jnp.where(di < HALF, 1, 0).astype(i32)

            return lax.fori_loop(0, ECH // 16, grp, cnt)

        cnt = lax.fori_loop(0, ET32 // ECH, chunk, jnp.zeros((16,), i32))
        cnt_v[...] = cnt
        pltpu.sync_copy(cnt_v, cnt_o.at[pl.ds(wid * 16, 16)])
        plsc.subcore_barrier()
        pltpu.sync_copy(acc.at[pl.ds(s * ECH, ECH)], z_v)
        pltpu.sync_copy(z_v, out.at[pl.ds(c * NPAD + s * ECH, ECH)])

    return body(dst, ew)


def _sc_norm(dinv, src, dst, ew):
    f32 = jnp.float32
    i32 = jnp.int32

    @functools.partial(
        pl.kernel,
        out_type=jax.ShapeDtypeStruct((EPAD,), f32),
        mesh=_mesh(),
        scratch_types=[
            pltpu.VMEM((NPAD,), f32),
            pltpu.VMEM((ECH,), i32),
            pltpu.VMEM((ECH,), i32),
            pltpu.VMEM((ECH,), f32),
            pltpu.VMEM((ECH,), f32),
        ],
        **_SCPARAMS,
    )
    def body(dinv_h, src_h, dst_h, ew_h, norm_o,
             dinv_v, src_v, dst_v, ew_v, nrm_v):
        c = lax.axis_index("c")
        s = lax.axis_index("s")
        wid = s * 2 + c
        pltpu.sync_copy(dinv_h, dinv_v)

        def chunk(j, _):
            base = wid * ET32 + j * ECH
            pltpu.sync_copy(src_h.at[pl.ds(base, ECH)], src_v)
            pltpu.sync_copy(dst_h.at[pl.ds(base, ECH)], dst_v)
            pltpu.sync_copy(ew_h.at[pl.ds(base, ECH)], ew_v)

            def grp(g, _):
                sl = pl.ds(g * 16, 16)
                a = plsc.load_gather(dinv_v, [src_v[sl]])
                b = plsc.load_gather(dinv_v, [dst_v[sl]])
                nrm_v[sl] = a * ew_v[sl] * b
                return 0

            lax.fori_loop(0, ECH // 16, grp, 0)
            pltpu.sync_copy(nrm_v, norm_o.at[pl.ds(base, ECH)])
            return 0

        lax.fori_loop(0, ET32 // ECH, chunk, 0)

    return body(dinv, src, dst, ew)


def _sc_part(src, dst, nrm, cnts):
    f32 = jnp.float32
    i32 = jnp.int32
    out_t = [jax.ShapeDtypeStruct((2 * EPAD,), i32),
             jax.ShapeDtypeStruct((2 * EPAD,), f32),
             jax.ShapeDtypeStruct((16,), i32)]

    @functools.partial(
        pl.kernel,
        out_type=out_t,
        mesh=_mesh(),
        scratch_types=[
            pltpu.VMEM((512,), i32),
            pltpu.VMEM((ECH,), i32),
            pltpu.VMEM((ECH,), i32),
            pltpu.VMEM((ECH,), f32),
            pltpu.VMEM((ECH,), i32),
            pltpu.VMEM((ECH,), i32),
            pltpu.VMEM_SHARED((PKSH,), i32),
            pltpu.VMEM_SHARED((PKSH,), f32),
        ],
        **_SCPARAMS,
    )
    def body(src_h, dst_h, nrm_h, cnts_h, pk_o, nm_o, c16_o,
             cv, src_v, dst_v, nrm_v, pk_v, pos_v, pk_sh, nm_sh):
        c = lax.axis_index("c")
        s = lax.axis_index("s")
        pltpu.sync_copy(cnts_h, cv)

        def pw(w, carry):
            tot, pre = carry
            rs = lax.reduce_sum(cv[pl.ds(w * 16, 16)], axes=(0,))
            pre = pre + jnp.where(w < 2 * s, rs, 0)
            return tot + rs, pre

        c0_total, s_pre = lax.fori_loop(0, 32, pw,
                                        (jnp.int32(0), jnp.int32(0)))
        my_off = jnp.where(c == 0, s_pre, 2 * s * ET32 - s_pre)
        dummy = EPAD + s * 16
        zslice = EPAD // 16

        def zb(i, _):
            pk_v[pl.ds(i * 16, 16)] = jnp.zeros((16,), i32)
            nrm_v[pl.ds(i * 16, 16)] = _zero16()
            return 0

        lax.fori_loop(0, ECH // 16, zb, 0)
        for q in range(zslice // ECH):
            pltpu.sync_copy(pk_v, pk_sh.at[pl.ds(s * zslice + q * ECH, ECH)])
            pltpu.sync_copy(nrm_v, nm_sh.at[pl.ds(s * zslice + q * ECH, ECH)])
        pltpu.sync_copy(pk_v.at[pl.ds(0, 16)], pk_sh.at[pl.ds(dummy, 16)])
        pltpu.sync_copy(nrm_v.at[pl.ds(0, 16)], nm_sh.at[pl.ds(dummy, 16)])
        plsc.subcore_barrier()

        def chunk(j, p):
            base = s * ET16 + j * ECH
            pltpu.sync_copy(src_h.at[pl.ds(base, ECH)], src_v)
            pltpu.sync_copy(dst_h.at[pl.ds(base, ECH)], dst_v)
            pltpu.sync_copy(nrm_h.at[pl.ds(base, ECH)], nrm_v)

            def grp(g, pp):
                sl = pl.ds(g * 16, 16)
                si = src_v[sl]
                di = dst_v[sl]
                m0 = di < HALF
                m = jnp.where(c == 0, m0, jnp.logical_not(m0))
                dl = jnp.where(m0, di, di - HALF)
                pk_v[sl] = jnp.bitwise_or(jnp.left_shift(si, 15), dl)
                ones = jnp.where(m, 1, 0).astype(i32)
                incl = plsc.cumsum(ones)
                pos_v[sl] = jnp.where(m, my_off + pp + incl - 1, dummy)
                return pp + lax.reduce_sum(ones, axes=(0,))

            p = lax.fori_loop(0, ECH // 16, grp, p)
            pltpu.sync_copy(pk_v, pk_sh.at[pos_v])
            pltpu.sync_copy(nrm_v, nm_sh.at[pos_v])
            return p

        lax.fori_loop(0, ET16 // ECH, chunk, jnp.int32(0))
        plsc.subcore_barrier()

        def ocp(q, _):
            o = s * zslice + q * ECH
            pltpu.sync_copy(pk_sh.at[pl.ds(o, ECH)], pk_v)
            pltpu.sync_copy(pk_v, pk_o.at[pl.ds(c * EPAD + o, ECH)])
            pltpu.sync_copy(nm_sh.at[pl.ds(o, ECH)], nrm_v)
            pltpu.sync_copy(nrm_v, nm_o.at[pl.ds(c * EPAD + o, ECH)])
            return 0

        lax.fori_loop(0, zslice // ECH, ocp, 0)

        @pl.when(jnp.logical_and(c == 0, s == 0))
        def _():
            it = _iota16()
            val = jnp.where(it == 0, c0_total,
                            jnp.where(it == 1, EPAD - c0_total, 0))
            pk_v[pl.ds(0, 16)] = val
            pltpu.sync_copy(pk_v.at[pl.ds(0, 16)], c16_o)

    return body(src, dst, nrm, cnts)


def _sc_hop_half(cur, pk, nm, c16, dw):
    f32 = jnp.float32
    i32 = jnp.int32

    @functools.partial(
        pl.kernel,
        out_type=jax.ShapeDtypeStruct((NPAD, dw), f32),
        mesh=_mesh(),
        scratch_types=[
            pltpu.VMEM((16,), i32),
            pltpu.VMEM((HCH,), i32), pltpu.VMEM((HCH,), f32),
            pltpu.VMEM((HCH,), i32), pltpu.VMEM((HCH,), i32),
            pltpu.VMEM((HCH,), i32), pltpu.VMEM((HCH,), f32),
            pltpu.VMEM((HCH,), i32), pltpu.VMEM((HCH,), i32),
            pltpu.VMEM((HCH, dw), f32), pltpu.VMEM((HCH, dw), f32),
            pltpu.VMEM_SHARED((HALF, dw), f32),
            pltpu.SemaphoreType.DMA, pltpu.SemaphoreType.DMA,
        ],
        **_SCPARAMS,
    )
    def body(cur_h, pk_h, nm_h, c16_h, out,
             cv, p0, n0, s0, d0, p1, n1, s1, d1, r0, r1, acc, g0, g1):
        c = lax.axis_index("c")
        s = lax.axis_index("s")
        pltpu.sync_copy(c16_h, cv)
        cnt = lax.reduce_max(jnp.where(_iota16() == c, cv[...], 0), axes=(0,))
        total = (cnt + HCH - 1) // HCH
        nloc = (total - s + 15) // 16

        def zr(i, _):
            for j in range(dw // 16):
                r0[i, pl.ds(j * 16, 16)] = _zero16()
            return 0

        lax.fori_loop(0, HCH, zr, 0)
        for q in range(3):
            pltpu.sync_copy(r0, acc.at[pl.ds(s * RH + q * 448, 448)])
        pltpu.sync_copy(r0.at[pl.ds(0, 224)],
                        acc.at[pl.ds(s * RH + 3 * 448, 224)])
        plsc.subcore_barrier()

        def fetch(i, pv, nv, sv, dv):
            b = c * EPAD + (s + i * 16) * HCH
            pltpu.sync_copy(pk_h.at[pl.ds(b, HCH)], pv)
            pltpu.sync_copy(nm_h.at[pl.ds(b, HCH)], nv)

            def unp(g, _):
                sl = pl.ds(g * 16, 16)
                w = pv[sl]
                sv[sl] = jnp.right_shift(w, 15)
                dv[sl] = jnp.bitwise_and(w, 32767)
                return 0

            lax.fori_loop(0, HCH // 16, unp, 0)

        def scale(rv, nv):
            def grp(g, _):
                nvv = nv[pl.ds(g * 16, 16)]
                for i in range(16):
                    e = g * 16 + i
                    bb = nvv.at[jnp.full((16,), i, i32)].get(
                        mode="promise_in_bounds")
                    for jj in range(dw // 16):
                        sl = pl.ds(jj * 16, 16)
                        rv[e, sl] = rv[e, sl] * bb
                return 0

            lax.fori_loop(0, HCH // 16, grp, 0)

        @pl.when(nloc > 0)
        def _():
            fetch(0, p0, n0, s0, d0)
            pltpu.async_copy(cur_h.at[s0], r0, g0)

        @pl.when(nloc > 1)
        def _():
            fetch(1, p1, n1, s1, d1)

        def it(i, _):
            q = i * 2
            pltpu.make_async_copy(cur_h.at[s0], r0, g0).wait()

            @pl.when(q + 1 < nloc)
            def _():
                pltpu.async_copy(cur_h.at[s1], r1, g1)

            scale(r0, n0)
            pltpu.sync_copy(r0, acc.at[d0], add=True)

            @pl.when(q + 2 < nloc)
            def _():
                fetch(q + 2, p0, n0, s0, d0)

            @pl.when(q + 1 < nloc)
            def _():
                pltpu.make_async_copy(cur_h.at[s1], r1, g1).wait()

                @pl.when(q + 2 < nloc)
                def _():
                    pltpu.async_copy(cur_h.at[s0], r0, g0)

                scale(r1, n1)
                pltpu.sync_copy(r1, acc.at[d1], add=True)

                @pl.when(q + 3 < nloc)
                def _():
                    fetch(q + 3, p1, n1, s1, d1)

            return 0

        lax.fori_loop(0, (nloc + 1) // 2, it, 0)
        plsc.subcore_barrier()
        for q in range(3):
            pltpu.sync_copy(acc.at[pl.ds(s * RH + q * 448, 448)], r0)
            pltpu.sync_copy(r0, out.at[pl.ds(c * HALF + s * RH + q * 448,
                                             448)])
        pltpu.sync_copy(acc.at[pl.ds(s * RH + 3 * 448, 224)],
                        r0.at[pl.ds(0, 224)])
        pltpu.sync_copy(r0.at[pl.ds(0, 224)],
                        out.at[pl.ds(c * HALF + s * RH + 3 * 448, 224)])

    return body(cur, pk, nm, c16)


def _sc_hop(curA, curB, pk, nm, c16):
    return (_sc_hop_half(curA, pk, nm, c16, DWA),
            _sc_hop_half(curB, pk, nm, c16, DWB))


def _elu(v):
    return jnp.where(v > 0, v, jnp.exp(jnp.minimum(v, 0.0)) - 1.0)


def _full2d(a):
    return pl.BlockSpec(a.shape, lambda i: (0,) * a.ndim)


def _tc_stage1(xp, idr, e0, e1, e2, d0, d1, p):
    f32 = jnp.float32

    def body(x_r, id_r, e0_r, e1_r, e2_r, d0_r, d1_r,
             wid_r, bid_r, w0_r, b0_r, we_r, be_r, g0_r, gb_r,
             ha_o, hb_o, dinv_o):
        idv = _elu(jnp.dot(id_r[...], wid_r[...],
                           preferred_element_type=f32) + bid_r[...])
        h0 = _elu(jnp.dot(x_r[...], w0_r[...],
                          preferred_element_type=f32) + b0_r[...])
        ecat = jnp.concatenate(
            [e0_r[...][:, :8], e1_r[...][:, :8], e2_r[...][:, :8]], axis=1)
        ev = _elu(jnp.dot(ecat, we_r[...],
                          preferred_element_type=f32) + be_r[...])
        hcat = jnp.concatenate([idv, h0, ev], axis=1)
        mu = jnp.mean(hcat, axis=1, keepdims=True)
        var = jnp.mean((hcat - mu) * (hcat - mu), axis=1, keepdims=True)
        hn = (hcat - mu) * lax.rsqrt(var + 1e-5) * g0_r[...] + gb_r[...]
        hp = jnp.concatenate([hn, jnp.zeros((256, DP - DREAL), f32)], axis=1)
        ha_o[...] = hp[:, :DWA]
        hb_o[...] = hp[:, DWA:]
        deg = d0_r[...] + d1_r[...]
        dinv_o[...] = jnp.where(deg > 0, lax.rsqrt(jnp.maximum(deg, 1e-30)),
                                0.0)

    wid = p['W_id']; bid = p['b_id'].reshape(1, -1)
    w0 = p['W0']; b0 = p['b0'].reshape(1, -1)
    we = p['W_emb']; be = p['b_emb'].reshape(1, -1)
    g0 = p['ln0_g'].reshape(1, -1); gb = p['ln0_b'].reshape(1, -1)
    row = lambda shp: pl.BlockSpec(shp, lambda i: (i, 0))
    row3 = pl.BlockSpec((1, 1, 256), lambda i: (i, 0, 0))
    return pl.pallas_call(
        body,
        grid=(GBLK,),
        in_specs=[row((256, 16)), row((256, 16)), row((256, 16)),
                  row((256, 16)), row((256, 16)), row3, row3,
                  _full2d(wid), _full2d(bid), _full2d(w0), _full2d(b0),
                  _full2d(we), _full2d(be), _full2d(g0), _full2d(gb)],
        out_specs=[row((256, DWA)), row((256, DWB)), row3],
        out_shape=[jax.ShapeDtypeStruct((NPAD, DWA), f32),
                   jax.ShapeDtypeStruct((NPAD, DWB), f32),
                   jax.ShapeDtypeStruct((GBLK, 1, 256), f32)],
    )(xp, idr, e0, e1, e2, d0, d1, wid, bid, w0, b0, we, be, g0, gb)


def _tc_final(halves, wall, p):
    f32 = jnp.float32

    def body(*refs):
        (ha_r, hb_r, c1a_r, c1b_r, c2a_r, c2b_r, c3a_r, c3b_r,
         wall_r, tb_r, g1_r, gb_r, w1_r, b1_r, o) = refs
        wr = wall_r[...]
        refs8 = (ha_r, hb_r, c1a_r, c1b_r, c2a_r, c2b_r, c3a_r, c3b_r)
        off = 0
        out = None
        for q, rr in enumerate(refs8):
            w = DWA if q % 2 == 0 else DWB
            term = jnp.dot(rr[...], wr[off:off + w, :],
                           preferred_element_type=f32)
            out = term if out is None else out + term
            off += w
        out = jnp.maximum(out + tb_r[...], 0.0)
        mu = jnp.mean(out, axis=1, keepdims=True)
        var = jnp.mean((out - mu) * (out - mu), axis=1, keepdims=True)
        out = (out - mu) * lax.rsqrt(var + 1e-5) * g1_r[...] + gb_r[...]
        y = jnp.dot(out, w1_r[...], preferred_element_type=f32) + b1_r[...]
        m = jnp.max(y, axis=1, keepdims=True)
        z = y - m
        o[...] = z - jnp.log(jnp.sum(jnp.exp(z), axis=1, keepdims=True))

    tb = p['tag_b'].reshape(1, -1)
    g1 = p['ln1_g'].reshape(1, -1); gb = p['ln1_b'].reshape(1, -1)
    w1 = p['W1']; b1 = p['b1'].reshape(1, -1)
    row = lambda shp: pl.BlockSpec(shp, lambda i: (i, 0))
    return pl.pallas_call(
        body,
        grid=(GBLK,),
        in_specs=[row((256, DWA)), row((256, DWB))] * 4 + [
            _full2d(wall), _full2d(tb),
            _full2d(g1), _full2d(gb), _full2d(w1), _full2d(b1)],
        out_specs=row((256, 2)),
        out_shape=jax.ShapeDtypeStruct((NPAD, 2), f32),
    )(*halves, wall, tb, g1, gb, w1, b1)


def kernel(x, edge_index, edge_weight, categories_value, params):
    p = params
    xp = jnp.pad(x, ((0, NPAD - N), (0, 0)))
    src = jnp.pad(edge_index[0], (0, EPAD - E))
    dst = jnp.pad(edge_index[1], (0, EPAD - E))
    ew = jnp.pad(edge_weight, (0, EPAD - E))
    cats = jnp.pad(categories_value.T.astype(jnp.int32),
                   ((0, 0), (0, NPAD - N))).reshape(4 * NPAD)
    embp = jnp.pad(p['emb_tables'], ((0, 0), (0, 0), (0, 8)))
    wall = jnp.pad(p['tag_W'], ((0, 0), (0, DP - DREAL), (0, 0)))
    wall = wall.reshape(4 * DP, -1)

    idr, e0r, e1r, e2r = _sc_gather(cats, p['id_table'], embp[0], embp[1],
                                    embp[2])
    deg2, cnts = _sc_degree(dst, ew)
    deg2 = deg2.reshape(2, NPAD)
    ha, hb, dinv3 = _tc_stage1(xp, idr, e0r, e1r, e2r,
                               deg2[0].reshape(GBLK, 1, 256),
                               deg2[1].reshape(GBLK, 1, 256), p)
    dinv = dinv3.reshape(NPAD)
    nrm = _sc_norm(dinv, src, dst, ew)
    pk, nm, c16 = _sc_part(src, dst, nrm, cnts)
    c1a, c1b = _sc_hop(ha, hb, pk, nm, c16)
    c2a, c2b = _sc_hop(c1a, c1b, pk, nm, c16)
    c3a, c3b = _sc_hop(c2a, c2b, pk, nm, c16)
    out = _tc_final((ha, hb, c1a, c1b, c2a, c2b, c3a, c3b), wall, p)
    return out[:N]


# speedup vs baseline: 12.6623x; 1.0505x over previous
import functools

import jax
import jax.numpy as jnp
from jax import lax
from jax.experimental import pallas as pl
from jax.experimental.pallas import tpu as pltpu
from jax.experimental.pallas import tpu_sc as plsc

N = 50000
NPAD = 50176
E = 800000
EPAD = 802816
DP = 80
DWA = 48
DWB = 32
DREAL = 72
HALF = NPAD // 2
RT = NPAD // 32
RH = HALF // 16
ET32 = EPAD // 32
ECH = 3136
ET16 = EPAD // 16
HCH = 320
GBLK = NPAD // 256
PKSH = EPAD + 256


def _mesh():
    return plsc.VectorSubcoreMesh(core_axis_name="c", subcore_axis_name="s",
                                  num_cores=2, num_subcores=16)


_SCPARAMS = dict(
    compiler_params=pltpu.CompilerParams(use_tc_tiling_on_sc=False,
                                         needs_layout_passes=False))


def _zero16():
    return jnp.zeros((16,), jnp.float32)


def _iota16():
    return lax.iota(jnp.int32, 16)


def _sc_gather(cats, id_table, emb0, emb1, emb2):
    f32 = jnp.float32
    out_t = [jax.ShapeDtypeStruct((NPAD, 16), f32) for _ in range(4)]

    @functools.partial(
        pl.kernel,
        out_type=out_t,
        mesh=_mesh(),
        scratch_types=[
            pltpu.VMEM((RT,), jnp.int32),
            pltpu.VMEM((RT, 16), f32),
            pltpu.SemaphoreType.DMA,
        ],
        **_SCPARAMS,
    )
    def body(cats_h, t0, t1, t2, t3, o0, o1, o2, o3, idx_v, rows_v, sem):
        c = lax.axis_index("c")
        s = lax.axis_index("s")
        base = (s * 2 + c) * RT
        for k, tbl, out in ((0, t0, o0), (1, t1, o1), (2, t2, o2), (3, t3, o3)):
            pltpu.sync_copy(cats_h.at[pl.ds(k * NPAD + base, RT)], idx_v)
            pltpu.async_copy(tbl.at[idx_v], rows_v, sem).wait()
            pltpu.sync_copy(rows_v, out.at[pl.ds(base, RT)])

    return body(cats, id_table, emb0, emb1, emb2)


def _sc_degree(dst, ew):
    f32 = jnp.float32
    i32 = jnp.int32
    out_t = [jax.ShapeDtypeStruct((2 * NPAD,), f32),
             jax.ShapeDtypeStruct((512,), i32)]

    @functools.partial(
        pl.kernel,
        out_type=out_t,
        mesh=_mesh(),
        scratch_types=[
            pltpu.VMEM((ECH,), i32),
            pltpu.VMEM((ECH,), f32),
            pltpu.VMEM((ECH,), f32),
            pltpu.VMEM((16,), i32),
            pltpu.VMEM_SHARED((NPAD,), f32),
        ],
        **_SCPARAMS,
    )
    def body(dst_h, ew_h, out, cnt_o, idx_v, val_v, z_v, cnt_v, acc):
        c = lax.axis_index("c")
        s = lax.axis_index("s")
        wid = s * 2 + c

        def zb(i, _):
            z_v[pl.ds(i * 16, 16)] = _zero16()
            return 0

        lax.fori_loop(0, ECH // 16, zb, 0)
        pltpu.sync_copy(z_v, acc.at[pl.ds(s * ECH, ECH)])
        plsc.subcore_barrier()

        def chunk(j, cnt):
            base = wid * ET32 + j * ECH
            pltpu.sync_copy(dst_h.at[pl.ds(base, ECH)], idx_v)
            pltpu.sync_copy(ew_h.at[pl.ds(base, ECH)], val_v)
            pltpu.sync_copy(val_v, acc.at[idx_v], add=True)

            def grp(g, cn):
                di = idx_v[pl.ds(g * 16, 16)]
                return cn + jnp.where(di < HALF, 1, 0).astype(i32)

            return lax.fori_loop(0, ECH // 16, grp, cnt)

        cnt = lax.fori_loop(0, ET32 // ECH, chunk, jnp.zeros((16,), i32))
        cnt_v[...] = cnt
        pltpu.sync_copy(cnt_v, cnt_o.at[pl.ds(wid * 16, 16)])
        plsc.subcore_barrier()
        pltpu.sync_copy(acc.at[pl.ds(s * ECH, ECH)], z_v)
        pltpu.sync_copy(z_v, out.at[pl.ds(c * NPAD + s * ECH, ECH)])

    return body(dst, ew)


def _sc_norm(dinv, src, dst, ew):
    f32 = jnp.float32
    i32 = jnp.int32

    @functools.partial(
        pl.kernel,
        out_type=jax.ShapeDtypeStruct((EPAD,), f32),
        mesh=_mesh(),
        scratch_types=[
            pltpu.VMEM((NPAD,), f32),
            pltpu.VMEM((ECH,), i32),
            pltpu.VMEM((ECH,), i32),
            pltpu.VMEM((ECH,), f32),
            pltpu.VMEM((ECH,), f32),
        ],
        **_SCPARAMS,
    )
    def body(dinv_h, src_h, dst_h, ew_h, norm_o,
             dinv_v, src_v, dst_v, ew_v, nrm_v):
        c = lax.axis_index("c")
        s = lax.axis_index("s")
        wid = s * 2 + c
        pltpu.sync_copy(dinv_h, dinv_v)

        def chunk(j, _):
            base = wid * ET32 + j * ECH
            pltpu.sync_copy(src_h.at[pl.ds(base, ECH)], src_v)
            pltpu.sync_copy(dst_h.at[pl.ds(base, ECH)], dst_v)
            pltpu.sync_copy(ew_h.at[pl.ds(base, ECH)], ew_v)

            def grp(g, _):
                sl = pl.ds(g * 16, 16)
                a = plsc.load_gather(dinv_v, [src_v[sl]])
                b = plsc.load_gather(dinv_v, [dst_v[sl]])
                nrm_v[sl] = a * ew_v[sl] * b
                return 0

            lax.fori_loop(0, ECH // 16, grp, 0)
            pltpu.sync_copy(nrm_v, norm_o.at[pl.ds(base, ECH)])
            return 0

        lax.fori_loop(0, ET32 // ECH, chunk, 0)

    return body(dinv, src, dst, ew)


def _sc_part(src, dst, nrm, cnts):
    f32 = jnp.float32
    i32 = jnp.int32
    out_t = [jax.ShapeDtypeStruct((2 * EPAD,), i32),
             jax.ShapeDtypeStruct((2 * EPAD,), f32),
             jax.ShapeDtypeStruct((16,), i32)]

    @functools.partial(
        pl.kernel,
        out_type=out_t,
        mesh=_mesh(),
        scratch_types=[
            pltpu.VMEM((512,), i32),
            pltpu.VMEM((ECH,), i32),
            pltpu.VMEM((ECH,), i32),
            pltpu.VMEM((ECH,), f32),
            pltpu.VMEM((ECH,), i32),
            pltpu.VMEM((ECH,), i32),
            pltpu.VMEM_SHARED((PKSH,), i32),
            pltpu.VMEM_SHARED((PKSH,), f32),
        ],
        **_SCPARAMS,
    )
    def body(src_h, dst_h, nrm_h, cnts_h, pk_o, nm_o, c16_o,
             cv, src_v, dst_v, nrm_v, pk_v, pos_v, pk_sh, nm_sh):
        c = lax.axis_index("c")
        s = lax.axis_index("s")
        pltpu.sync_copy(cnts_h, cv)

        def pw(w, carry):
            tot, pre = carry
            rs = lax.reduce_sum(cv[pl.ds(w * 16, 16)], axes=(0,))
            pre = pre + jnp.where(w < 2 * s, rs, 0)
            return tot + rs, pre

        c0_total, s_pre = lax.fori_loop(0, 32, pw,
                                        (jnp.int32(0), jnp.int32(0)))
        my_off = jnp.where(c == 0, s_pre, 2 * s * ET32 - s_pre)
        dummy = EPAD + s * 16
        zslice = EPAD // 16

        def zb(i, _):
            pk_v[pl.ds(i * 16, 16)] = jnp.zeros((16,), i32)
            nrm_v[pl.ds(i * 16, 16)] = _zero16()
            return 0

        lax.fori_loop(0, ECH // 16, zb, 0)
        for q in range(zslice // ECH):
            pltpu.sync_copy(pk_v, pk_sh.at[pl.ds(s * zslice + q * ECH, ECH)])
            pltpu.sync_copy(nrm_v, nm_sh.at[pl.ds(s * zslice + q * ECH, ECH)])
        pltpu.sync_copy(pk_v.at[pl.ds(0, 16)], pk_sh.at[pl.ds(dummy, 16)])
        pltpu.sync_copy(nrm_v.at[pl.ds(0, 16)], nm_sh.at[pl.ds(dummy, 16)])
        plsc.subcore_barrier()

        def chunk(j, p):
            base = s * ET16 + j * ECH
            pltpu.sync_copy(src_h.at[pl.ds(base, ECH)], src_v)
            pltpu.sync_copy(dst_h.at[pl.ds(base, ECH)], dst_v)
            pltpu.sync_copy(nrm_h.at[pl.ds(base, ECH)], nrm_v)

            def grp(g, pp):
                sl = pl.ds(g * 16, 16)
                si = src_v[sl]
                di = dst_v[sl]
                m0 = di < HALF
                m = jnp.where(c == 0, m0, jnp.logical_not(m0))
                dl = jnp.where(m0, di, di - HALF)
                pk_v[sl] = jnp.bitwise_or(jnp.left_shift(si, 15), dl)
                ones = jnp.where(m, 1, 0).astype(i32)
                incl = plsc.cumsum(ones)
                pos_v[sl] = jnp.where(m, my_off + pp + incl - 1, dummy)
                return pp + lax.reduce_sum(ones, axes=(0,))

            p = lax.fori_loop(0, ECH // 16, grp, p)
            pltpu.sync_copy(pk_v, pk_sh.at[pos_v])
            pltpu.sync_copy(nrm_v, nm_sh.at[pos_v])
            return p

        lax.fori_loop(0, ET16 // ECH, chunk, jnp.int32(0))
        plsc.subcore_barrier()

        def ocp(q, _):
            o = s * zslice + q * ECH
            pltpu.sync_copy(pk_sh.at[pl.ds(o, ECH)], pk_v)
            pltpu.sync_copy(pk_v, pk_o.at[pl.ds(c * EPAD + o, ECH)])
            pltpu.sync_copy(nm_sh.at[pl.ds(o, ECH)], nrm_v)
            pltpu.sync_copy(nrm_v, nm_o.at[pl.ds(c * EPAD + o, ECH)])
            return 0

        lax.fori_loop(0, zslice // ECH, ocp, 0)

        @pl.when(jnp.logical_and(c == 0, s == 0))
        def _():
            it = _iota16()
            val = jnp.where(it == 0, c0_total,
                            jnp.where(it == 1, EPAD - c0_total, 0))
            pk_v[pl.ds(0, 16)] = val
            pltpu.sync_copy(pk_v.at[pl.ds(0, 16)], c16_o)

    return body(src, dst, nrm, cnts)


def _sc_hop_half(cur, pk, nm, c16, dw):
    f32 = jnp.float32
    i32 = jnp.int32

    @functools.partial(
        pl.kernel,
        out_type=jax.ShapeDtypeStruct((NPAD, dw), f32),
        mesh=_mesh(),
        scratch_types=[
            pltpu.VMEM((16,), i32),
            pltpu.VMEM((HCH,), i32), pltpu.VMEM((HCH,), f32),
            pltpu.VMEM((HCH,), i32), pltpu.VMEM((HCH,), i32),
            pltpu.VMEM((HCH,), i32), pltpu.VMEM((HCH,), f32),
            pltpu.VMEM((HCH,), i32), pltpu.VMEM((HCH,), i32),
            pltpu.VMEM((HCH,), i32), pltpu.VMEM((HCH,), f32),
            pltpu.VMEM((HCH,), i32), pltpu.VMEM((HCH,), i32),
            pltpu.VMEM((HCH, dw), f32), pltpu.VMEM((HCH, dw), f32),
            pltpu.VMEM((HCH, dw), f32),
            pltpu.VMEM_SHARED((HALF, dw), f32),
            pltpu.SemaphoreType.DMA, pltpu.SemaphoreType.DMA,
            pltpu.SemaphoreType.DMA, pltpu.SemaphoreType.DMA,
            pltpu.SemaphoreType.DMA, pltpu.SemaphoreType.DMA,
        ],
        **_SCPARAMS,
    )
    def body(cur_h, pk_h, nm_h, c16_h, out, cv,
             p0, n0, s0, d0, p1, n1, s1, d1, p2, n2, s2, d2,
             r0, r1, r2, acc, g0, g1, g2, c0, c1, c2):
        c = lax.axis_index("c")
        s = lax.axis_index("s")
        pltpu.sync_copy(c16_h, cv)
        cnt = lax.reduce_max(jnp.where(_iota16() == c, cv[...], 0), axes=(0,))
        total = (cnt + HCH - 1) // HCH
        nloc = (total - s + 15) // 16
        P = ((p0, n0, s0, d0, r0, g0, c0),
             (p1, n1, s1, d1, r1, g1, c1),
             (p2, n2, s2, d2, r2, g2, c2))

        def zr(i, _):
            for j in range(dw // 16):
                r0[i, pl.ds(j * 16, 16)] = _zero16()
            return 0

        lax.fori_loop(0, HCH, zr, 0)
        for q in range(7):
            pltpu.sync_copy(r0.at[pl.ds(0, 224)],
                            acc.at[pl.ds(s * RH + q * 224, 224)])
        plsc.subcore_barrier()

        def fetch(i, pv, nv, sv, dv):
            b = c * EPAD + (s + i * 16) * HCH
            pltpu.sync_copy(pk_h.at[pl.ds(b, HCH)], pv)
            pltpu.sync_copy(nm_h.at[pl.ds(b, HCH)], nv)

            def unp(g, _):
                sl = pl.ds(g * 16, 16)
                w = pv[sl]
                sv[sl] = jnp.right_shift(w, 15)
                dv[sl] = jnp.bitwise_and(w, 32767)
                return 0

            lax.fori_loop(0, HCH // 16, unp, 0)

        def scale(rv, nv):
            def grp(g, _):
                nvv = nv[pl.ds(g * 16, 16)]
                for i in range(16):
                    e = g * 16 + i
                    bb = nvv.at[jnp.full((16,), i, i32)].get(
                        mode="promise_in_bounds")
                    for jj in range(dw // 16):
                        sl = pl.ds(jj * 16, 16)
                        rv[e, sl] = rv[e, sl] * bb
                return 0

            lax.fori_loop(0, HCH // 16, grp, 0)

        for b in range(2):
            pv, nv, sv, dv, rv, gv, _cv = P[b]

            @pl.when(nloc > b)
            def _(b=b, pv=pv, nv=nv, sv=sv, dv=dv, rv=rv, gv=gv):
                fetch(b, pv, nv, sv, dv)
                pltpu.async_copy(cur_h.at[sv], rv, gv)

        def it(i, _):
            for b in range(3):
                q = i * 3 + b
                pv, nv, sv, dv, rv, gv, scv = P[b]
                b2 = (b + 2) % 3
                pv2, nv2, sv2, dv2, rv2, gv2, scv2 = P[b2]

                @pl.when(q < nloc)
                def _(q=q, pv=pv, nv=nv, sv=sv, dv=dv, rv=rv, gv=gv,
                      scv=scv, pv2=pv2, nv2=nv2, sv2=sv2, dv2=dv2,
                      rv2=rv2, gv2=gv2, scv2=scv2):
                    pltpu.make_async_copy(cur_h.at[sv], rv, gv).wait()
                    scale(rv, nv)
                    pltpu.async_copy(rv, acc.at[dv], scv, add=True)

                    @pl.when(q >= 1)
                    def _():
                        pltpu.make_async_copy(rv2, acc.at[dv2], scv2).wait()

                    @pl.when(q + 2 < nloc)
                    def _():
                        fetch(q + 2, pv2, nv2, sv2, dv2)
                        pltpu.async_copy(cur_h.at[sv2], rv2, gv2)

            return 0

        lax.fori_loop(0, (nloc + 2) // 3, it, 0)
        for b in range(3):
            pv, nv, sv, dv, rv, gv, scv = P[b]
            last1 = jnp.logical_and(nloc >= 1, lax.rem(nloc - 1, 3) == b)

            @pl.when(last1)
            def _(rv=rv, dv=dv, scv=scv):
                pltpu.make_async_copy(rv, acc.at[dv], scv).wait()

        plsc.subcore_barrier()
        for q in range(7):
            pltpu.sync_copy(acc.at[pl.ds(s * RH + q * 224, 224)],
                            r0.at[pl.ds(0, 224)])
            pltpu.sync_copy(r0.at[pl.ds(0, 224)],
                            out.at[pl.ds(c * HALF + s * RH + q * 224, 224)])

    return body(cur, pk, nm, c16)


def _sc_hop(curA, curB, pk, nm, c16):
    return (_sc_hop_half(curA, pk, nm, c16, DWA),
            _sc_hop_half(curB, pk, nm, c16, DWB))


def _elu(v):
    return jnp.where(v > 0, v, jnp.exp(jnp.minimum(v, 0.0)) - 1.0)


def _full2d(a):
    return pl.BlockSpec(a.shape, lambda i: (0,) * a.ndim)


def _tc_stage1(xp, idr, e0, e1, e2, d0, d1, p):
    f32 = jnp.float32

    def body(x_r, id_r, e0_r, e1_r, e2_r, d0_r, d1_r,
             wid_r, bid_r, w0_r, b0_r, we_r, be_r, g0_r, gb_r,
             ha_o, hb_o, dinv_o):
        idv = _elu(jnp.dot(id_r[...], wid_r[...],
                           preferred_element_type=f32) + bid_r[...])
        h0 = _elu(jnp.dot(x_r[...], w0_r[...],
                          preferred_element_type=f32) + b0_r[...])
        ecat = jnp.concatenate(
            [e0_r[...][:, :8], e1_r[...][:, :8], e2_r[...][:, :8]], axis=1)
        ev = _elu(jnp.dot(ecat, we_r[...],
                          preferred_element_type=f32) + be_r[...])
        hcat = jnp.concatenate([idv, h0, ev], axis=1)
        mu = jnp.mean(hcat, axis=1, keepdims=True)
        var = jnp.mean((hcat - mu) * (hcat - mu), axis=1, keepdims=True)
        hn = (hcat - mu) * lax.rsqrt(var + 1e-5) * g0_r[...] + gb_r[...]
        hp = jnp.concatenate([hn, jnp.zeros((256, DP - DREAL), f32)], axis=1)
        ha_o[...] = hp[:, :DWA]
        hb_o[...] = hp[:, DWA:]
        deg = d0_r[...] + d1_r[...]
        dinv_o[...] = jnp.where(deg > 0, lax.rsqrt(jnp.maximum(deg, 1e-30)),
                                0.0)

    wid = p['W_id']; bid = p['b_id'].reshape(1, -1)
    w0 = p['W0']; b0 = p['b0'].reshape(1, -1)
    we = p['W_emb']; be = p['b_emb'].reshape(1, -1)
    g0 = p['ln0_g'].reshape(1, -1); gb = p['ln0_b'].reshape(1, -1)
    row = lambda shp: pl.BlockSpec(shp, lambda i: (i, 0))
    row3 = pl.BlockSpec((1, 1, 256), lambda i: (i, 0, 0))
    return pl.pallas_call(
        body,
        grid=(GBLK,),
        in_specs=[row((256, 16)), row((256, 16)), row((256, 16)),
                  row((256, 16)), row((256, 16)), row3, row3,
                  _full2d(wid), _full2d(bid), _full2d(w0), _full2d(b0),
                  _full2d(we), _full2d(be), _full2d(g0), _full2d(gb)],
        out_specs=[row((256, DWA)), row((256, DWB)), row3],
        out_shape=[jax.ShapeDtypeStruct((NPAD, DWA), f32),
                   jax.ShapeDtypeStruct((NPAD, DWB), f32),
                   jax.ShapeDtypeStruct((GBLK, 1, 256), f32)],
    )(xp, idr, e0, e1, e2, d0, d1, wid, bid, w0, b0, we, be, g0, gb)


def _tc_final(halves, wall, p):
    f32 = jnp.float32

    def body(*refs):
        (ha_r, hb_r, c1a_r, c1b_r, c2a_r, c2b_r, c3a_r, c3b_r,
         wall_r, tb_r, g1_r, gb_r, w1_r, b1_r, o) = refs
        wr = wall_r[...]
        refs8 = (ha_r, hb_r, c1a_r, c1b_r, c2a_r, c2b_r, c3a_r, c3b_r)
        off = 0
        out = None
        for q, rr in enumerate(refs8):
            w = DWA if q % 2 == 0 else DWB
            term = jnp.dot(rr[...], wr[off:off + w, :],
                           preferred_element_type=f32)
            out = term if out is None else out + term
            off += w
        out = jnp.maximum(out + tb_r[...], 0.0)
        mu = jnp.mean(out, axis=1, keepdims=True)
        var = jnp.mean((out - mu) * (out - mu), axis=1, keepdims=True)
        out = (out - mu) * lax.rsqrt(var + 1e-5) * g1_r[...] + gb_r[...]
        y = jnp.dot(out, w1_r[...], preferred_element_type=f32) + b1_r[...]
        m = jnp.max(y, axis=1, keepdims=True)
        z = y - m
        o[...] = z - jnp.log(jnp.sum(jnp.exp(z), axis=1, keepdims=True))

    tb = p['tag_b'].reshape(1, -1)
    g1 = p['ln1_g'].reshape(1, -1); gb = p['ln1_b'].reshape(1, -1)
    w1 = p['W1']; b1 = p['b1'].reshape(1, -1)
    row = lambda shp: pl.BlockSpec(shp, lambda i: (i, 0))
    return pl.pallas_call(
        body,
        grid=(GBLK,),
        in_specs=[row((256, DWA)), row((256, DWB))] * 4 + [
            _full2d(wall), _full2d(tb),
            _full2d(g1), _full2d(gb), _full2d(w1), _full2d(b1)],
        out_specs=row((256, 2)),
        out_shape=jax.ShapeDtypeStruct((NPAD, 2), f32),
    )(*halves, wall, tb, g1, gb, w1, b1)


def kernel(x, edge_index, edge_weight, categories_value, params):
    p = params
    xp = jnp.pad(x, ((0, NPAD - N), (0, 0)))
    src = jnp.pad(edge_index[0], (0, EPAD - E))
    dst = jnp.pad(edge_index[1], (0, EPAD - E))
    ew = jnp.pad(edge_weight, (0, EPAD - E))
    cats = jnp.pad(categories_value.T.astype(jnp.int32),
                   ((0, 0), (0, NPAD - N))).reshape(4 * NPAD)
    embp = jnp.pad(p['emb_tables'], ((0, 0), (0, 0), (0, 8)))
    wall = jnp.pad(p['tag_W'], ((0, 0), (0, DP - DREAL), (0, 0)))
    wall = wall.reshape(4 * DP, -1)

    idr, e0r, e1r, e2r = _sc_gather(cats, p['id_table'], embp[0], embp[1],
                                    embp[2])
    deg2, cnts = _sc_degree(dst, ew)
    deg2 = deg2.reshape(2, NPAD)
    ha, hb, dinv3 = _tc_stage1(xp, idr, e0r, e1r, e2r,
                               deg2[0].reshape(GBLK, 1, 256),
                               deg2[1].reshape(GBLK, 1, 256), p)
    dinv = dinv3.reshape(NPAD)
    nrm = _sc_norm(dinv, src, dst, ew)
    pk, nm, c16 = _sc_part(src, dst, nrm, cnts)
    c1a, c1b = _sc_hop(ha, hb, pk, nm, c16)
    c2a, c2b = _sc_hop(c1a, c1b, pk, nm, c16)
    c3a, c3b = _sc_hop(c2a, c2b, pk, nm, c16)
    out = _tc_final((ha, hb, c1a, c1b, c2a, c2b, c3a, c3b), wall, p)
    return out[:N]


# speedup vs baseline: 13.0513x; 1.0307x over previous
import functools

import jax
import jax.numpy as jnp
from jax import lax
from jax.experimental import pallas as pl
from jax.experimental.pallas import tpu as pltpu
from jax.experimental.pallas import tpu_sc as plsc

N = 50000
NPAD = 50176
E = 800000
EPAD = 802816
DP = 80
DWA = 48
DWB = 32
DREAL = 72
HALF = NPAD // 2
RT = NPAD // 32
RH = HALF // 16
ET32 = EPAD // 32
ECH = 3136
ET16 = EPAD // 16
HCHA = 320
HCHB = 512
GBLK = NPAD // 256
PKSH = EPAD + 256


def _mesh():
    return plsc.VectorSubcoreMesh(core_axis_name="c", subcore_axis_name="s",
                                  num_cores=2, num_subcores=16)


_SCPARAMS = dict(
    compiler_params=pltpu.CompilerParams(use_tc_tiling_on_sc=False,
                                         needs_layout_passes=False))


def _zero16():
    return jnp.zeros((16,), jnp.float32)


def _iota16():
    return lax.iota(jnp.int32, 16)


def _sc_gather(cats, id_table, emb0, emb1, emb2):
    f32 = jnp.float32
    out_t = [jax.ShapeDtypeStruct((NPAD, 16), f32) for _ in range(4)]

    @functools.partial(
        pl.kernel,
        out_type=out_t,
        mesh=_mesh(),
        scratch_types=[
            pltpu.VMEM((RT,), jnp.int32),
            pltpu.VMEM((RT, 16), f32),
            pltpu.SemaphoreType.DMA,
        ],
        **_SCPARAMS,
    )
    def body(cats_h, t0, t1, t2, t3, o0, o1, o2, o3, idx_v, rows_v, sem):
        c = lax.axis_index("c")
        s = lax.axis_index("s")
        base = (s * 2 + c) * RT
        for k, tbl, out in ((0, t0, o0), (1, t1, o1), (2, t2, o2), (3, t3, o3)):
            pltpu.sync_copy(cats_h.at[pl.ds(k * NPAD + base, RT)], idx_v)
            pltpu.async_copy(tbl.at[idx_v], rows_v, sem).wait()
            pltpu.sync_copy(rows_v, out.at[pl.ds(base, RT)])

    return body(cats, id_table, emb0, emb1, emb2)


def _sc_degree(dst, ew):
    f32 = jnp.float32
    i32 = jnp.int32
    out_t = [jax.ShapeDtypeStruct((2 * NPAD,), f32),
             jax.ShapeDtypeStruct((512,), i32)]

    @functools.partial(
        pl.kernel,
        out_type=out_t,
        mesh=_mesh(),
        scratch_types=[
            pltpu.VMEM((ECH,), i32),
            pltpu.VMEM((ECH,), f32),
            pltpu.VMEM((ECH,), f32),
            pltpu.VMEM((16,), i32),
            pltpu.VMEM_SHARED((NPAD,), f32),
        ],
        **_SCPARAMS,
    )
    def body(dst_h, ew_h, out, cnt_o, idx_v, val_v, z_v, cnt_v, acc):
        c = lax.axis_index("c")
        s = lax.axis_index("s")
        wid = s * 2 + c

        def zb(i, _):
            z_v[pl.ds(i * 16, 16)] = _zero16()
            return 0

        lax.fori_loop(0, ECH // 16, zb, 0)
        pltpu.sync_copy(z_v, acc.at[pl.ds(s * ECH, ECH)])
        plsc.subcore_barrier()

        def chunk(j, cnt):
            base = wid * ET32 + j * ECH
            pltpu.sync_copy(dst_h.at[pl.ds(base, ECH)], idx_v)
            pltpu.sync_copy(ew_h.at[pl.ds(base, ECH)], val_v)
            pltpu.sync_copy(val_v, acc.at[idx_v], add=True)

            def grp(g, cn):
                di = idx_v[pl.ds(g * 16, 16)]
                return cn + jnp.where(di < HALF, 1, 0).astype(i32)

            return lax.fori_loop(0, ECH // 16, grp, cnt)

        cnt = lax.fori_loop(0, ET32 // ECH, chunk, jnp.zeros((16,), i32))
        cnt_v[...] = cnt
        pltpu.sync_copy(cnt_v, cnt_o.at[pl.ds(wid * 16, 16)])
        plsc.subcore_barrier()
        pltpu.sync_copy(acc.at[pl.ds(s * ECH, ECH)], z_v)
        pltpu.sync_copy(z_v, out.at[pl.ds(c * NPAD + s * ECH, ECH)])

    return body(dst, ew)


def _sc_norm(dinv, src, dst, ew):
    f32 = jnp.float32
    i32 = jnp.int32

    @functools.partial(
        pl.kernel,
        out_type=jax.ShapeDtypeStruct((EPAD,), f32),
        mesh=_mesh(),
        scratch_types=[
            pltpu.VMEM((NPAD,), f32),
            pltpu.VMEM((ECH,), i32),
            pltpu.VMEM((ECH,), i32),
            pltpu.VMEM((ECH,), f32),
            pltpu.VMEM((ECH,), f32),
            pltpu.VMEM_SHARED((NPAD,), f32),
        ],
        **_SCPARAMS,
    )
    def body(dinv_h, src_h, dst_h, ew_h, norm_o,
             dinv_v, src_v, dst_v, ew_v, nrm_v, dinv_sh):
        c = lax.axis_index("c")
        s = lax.axis_index("s")
        wid = s * 2 + c
        pltpu.sync_copy(dinv_h.at[pl.ds(s * ECH, ECH)],
                        dinv_v.at[pl.ds(s * ECH, ECH)])
        pltpu.sync_copy(dinv_v.at[pl.ds(s * ECH, ECH)],
                        dinv_sh.at[pl.ds(s * ECH, ECH)])
        plsc.subcore_barrier()
        pltpu.sync_copy(dinv_sh, dinv_v)

        def chunk(j, _):
            base = wid * ET32 + j * ECH
            pltpu.sync_copy(src_h.at[pl.ds(base, ECH)], src_v)
            pltpu.sync_copy(dst_h.at[pl.ds(base, ECH)], dst_v)
            pltpu.sync_copy(ew_h.at[pl.ds(base, ECH)], ew_v)

            def grp(g, _):
                sl = pl.ds(g * 16, 16)
                a = plsc.load_gather(dinv_v, [src_v[sl]])
                b = plsc.load_gather(dinv_v, [dst_v[sl]])
                nrm_v[sl] = a * ew_v[sl] * b
                return 0

            lax.fori_loop(0, ECH // 16, grp, 0)
            pltpu.sync_copy(nrm_v, norm_o.at[pl.ds(base, ECH)])
            return 0

        lax.fori_loop(0, ET32 // ECH, chunk, 0)

    return body(dinv, src, dst, ew)


def _sc_part(src, dst, nrm, cnts):
    f32 = jnp.float32
    i32 = jnp.int32
    out_t = [jax.ShapeDtypeStruct((2 * EPAD,), i32),
             jax.ShapeDtypeStruct((2 * EPAD,), f32),
             jax.ShapeDtypeStruct((16,), i32)]

    @functools.partial(
        pl.kernel,
        out_type=out_t,
        mesh=_mesh(),
        scratch_types=[
            pltpu.VMEM((512,), i32),
            pltpu.VMEM((ECH,), i32),
            pltpu.VMEM((ECH,), i32),
            pltpu.VMEM((ECH,), f32),
            pltpu.VMEM((ECH,), i32),
            pltpu.VMEM((ECH,), i32),
            pltpu.VMEM_SHARED((PKSH,), i32),
            pltpu.VMEM_SHARED((PKSH,), f32),
        ],
        **_SCPARAMS,
    )
    def body(src_h, dst_h, nrm_h, cnts_h, pk_o, nm_o, c16_o,
             cv, src_v, dst_v, nrm_v, pk_v, pos_v, pk_sh, nm_sh):
        c = lax.axis_index("c")
        s = lax.axis_index("s")
        pltpu.sync_copy(cnts_h, cv)

        def pw(w, carry):
            tot, pre = carry
            rs = lax.reduce_sum(cv[pl.ds(w * 16, 16)], axes=(0,))
            pre = pre + jnp.where(w < 2 * s, rs, 0)
            return tot + rs, pre

        c0_total, s_pre = lax.fori_loop(0, 32, pw,
                                        (jnp.int32(0), jnp.int32(0)))
        my_off = jnp.where(c == 0, s_pre, 2 * s * ET32 - s_pre)
        dummy = EPAD + s * 16
        zslice = EPAD // 16

        def zb(i, _):
            pk_v[pl.ds(i * 16, 16)] = jnp.zeros((16,), i32)
            nrm_v[pl.ds(i * 16, 16)] = _zero16()
            return 0

        lax.fori_loop(0, ECH // 16, zb, 0)
        for q in range(zslice // ECH):
            pltpu.sync_copy(pk_v, pk_sh.at[pl.ds(s * zslice + q * ECH, ECH)])
            pltpu.sync_copy(nrm_v, nm_sh.at[pl.ds(s * zslice + q * ECH, ECH)])
        pltpu.sync_copy(pk_v.at[pl.ds(0, 16)], pk_sh.at[pl.ds(dummy, 16)])
        pltpu.sync_copy(nrm_v.at[pl.ds(0, 16)], nm_sh.at[pl.ds(dummy, 16)])
        plsc.subcore_barrier()

        def chunk(j, p):
            base = s * ET16 + j * ECH
            pltpu.sync_copy(src_h.at[pl.ds(base, ECH)], src_v)
            pltpu.sync_copy(dst_h.at[pl.ds(base, ECH)], dst_v)
            pltpu.sync_copy(nrm_h.at[pl.ds(base, ECH)], nrm_v)

            def grp(g, pp):
                sl = pl.ds(g * 16, 16)
                si = src_v[sl]
                di = dst_v[sl]
                m0 = di < HALF
                m = jnp.where(c == 0, m0, jnp.logical_not(m0))
                dl = jnp.where(m0, di, di - HALF)
                pk_v[sl] = jnp.bitwise_or(jnp.left_shift(si, 15), dl)
                ones = jnp.where(m, 1, 0).astype(i32)
                incl = plsc.cumsum(ones)
                pos_v[sl] = jnp.where(m, my_off + pp + incl - 1, dummy)
                return pp + lax.reduce_sum(ones, axes=(0,))

            p = lax.fori_loop(0, ECH // 16, grp, p)
            pltpu.sync_copy(pk_v, pk_sh.at[pos_v])
            pltpu.sync_copy(nrm_v, nm_sh.at[pos_v])
            return p

        lax.fori_loop(0, ET16 // ECH, chunk, jnp.int32(0))
        plsc.subcore_barrier()

        def ocp(q, _):
            o = s * zslice + q * ECH
            pltpu.sync_copy(pk_sh.at[pl.ds(o, ECH)], pk_v)
            pltpu.sync_copy(pk_v, pk_o.at[pl.ds(c * EPAD + o, ECH)])
            pltpu.sync_copy(nm_sh.at[pl.ds(o, ECH)], nrm_v)
            pltpu.sync_copy(nrm_v, nm_o.at[pl.ds(c * EPAD + o, ECH)])
            return 0

        lax.fori_loop(0, zslice // ECH, ocp, 0)

        @pl.when(jnp.logical_and(c == 0, s == 0))
        def _():
            it = _iota16()
            val = jnp.where(it == 0, c0_total,
                            jnp.where(it == 1, EPAD - c0_total, 0))
            pk_v[pl.ds(0, 16)] = val
            pltpu.sync_copy(pk_v.at[pl.ds(0, 16)], c16_o)

    return body(src, dst, nrm, cnts)


def _sc_hop_half(cur, pk, nm, c16, dw):
    f32 = jnp.float32
    i32 = jnp.int32
    HCH = HCHA if dw == DWA else HCHB

    @functools.partial(
        pl.kernel,
        out_type=jax.ShapeDtypeStruct((NPAD, dw), f32),
        mesh=_mesh(),
        scratch_types=[
            pltpu.VMEM((16,), i32),
            pltpu.VMEM((HCH,), i32), pltpu.VMEM((HCH,), f32),
            pltpu.VMEM((HCH,), i32), pltpu.VMEM((HCH,), i32),
            pltpu.VMEM((HCH,), i32), pltpu.VMEM((HCH,), f32),
            pltpu.VMEM((HCH,), i32), pltpu.VMEM((HCH,), i32),
            pltpu.VMEM((HCH,), i32), pltpu.VMEM((HCH,), f32),
            pltpu.VMEM((HCH,), i32), pltpu.VMEM((HCH,), i32),
            pltpu.VMEM((HCH, dw), f32), pltpu.VMEM((HCH, dw), f32),
            pltpu.VMEM((HCH, dw), f32),
            pltpu.VMEM_SHARED((HALF, dw), f32),
            pltpu.SemaphoreType.DMA, pltpu.SemaphoreType.DMA,
            pltpu.SemaphoreType.DMA, pltpu.SemaphoreType.DMA,
            pltpu.SemaphoreType.DMA, pltpu.SemaphoreType.DMA,
        ],
        **_SCPARAMS,
    )
    def body(cur_h, pk_h, nm_h, c16_h, out, cv,
             p0, n0, s0, d0, p1, n1, s1, d1, p2, n2, s2, d2,
             r0, r1, r2, acc, g0, g1, g2, c0, c1, c2):
        c = lax.axis_index("c")
        s = lax.axis_index("s")
        pltpu.sync_copy(c16_h, cv)
        cnt = lax.reduce_max(jnp.where(_iota16() == c, cv[...], 0), axes=(0,))
        total = (cnt + HCH - 1) // HCH
        nloc = (total - s + 15) // 16
        P = ((p0, n0, s0, d0, r0, g0, c0),
             (p1, n1, s1, d1, r1, g1, c1),
             (p2, n2, s2, d2, r2, g2, c2))

        def zr(i, _):
            for j in range(dw // 16):
                r0[i, pl.ds(j * 16, 16)] = _zero16()
            return 0

        lax.fori_loop(0, HCH, zr, 0)
        for q in range(7):
            pltpu.sync_copy(r0.at[pl.ds(0, 224)],
                            acc.at[pl.ds(s * RH + q * 224, 224)])
        plsc.subcore_barrier()

        def fetch(i, pv, nv, sv, dv):
            b = c * EPAD + (s + i * 16) * HCH
            pltpu.sync_copy(pk_h.at[pl.ds(b, HCH)], pv)
            pltpu.sync_copy(nm_h.at[pl.ds(b, HCH)], nv)

            def unp(g, _):
                sl = pl.ds(g * 16, 16)
                w = pv[sl]
                sv[sl] = jnp.right_shift(w, 15)
                dv[sl] = jnp.bitwise_and(w, 32767)
                return 0

            lax.fori_loop(0, HCH // 16, unp, 0)

        def scale(rv, nv):
            def grp(g, _):
                nvv = nv[pl.ds(g * 16, 16)]
                for i in range(16):
                    e = g * 16 + i
                    bb = nvv.at[jnp.full((16,), i, i32)].get(
                        mode="promise_in_bounds")
                    for jj in range(dw // 16):
                        sl = pl.ds(jj * 16, 16)
                        rv[e, sl] = rv[e, sl] * bb
                return 0

            lax.fori_loop(0, HCH // 16, grp, 0)

        for b in range(2):
            pv, nv, sv, dv, rv, gv, _cv = P[b]

            @pl.when(nloc > b)
            def _(b=b, pv=pv, nv=nv, sv=sv, dv=dv, rv=rv, gv=gv):
                fetch(b, pv, nv, sv, dv)
                pltpu.async_copy(cur_h.at[sv], rv, gv)

        def it(i, _):
            for b in range(3):
                q = i * 3 + b
                pv, nv, sv, dv, rv, gv, scv = P[b]
                b2 = (b + 2) % 3
                pv2, nv2, sv2, dv2, rv2, gv2, scv2 = P[b2]

                @pl.when(q < nloc)
                def _(q=q, pv=pv, nv=nv, sv=sv, dv=dv, rv=rv, gv=gv,
                      scv=scv, pv2=pv2, nv2=nv2, sv2=sv2, dv2=dv2,
                      rv2=rv2, gv2=gv2, scv2=scv2):
                    pltpu.make_async_copy(cur_h.at[sv], rv, gv).wait()
                    scale(rv, nv)
                    pltpu.async_copy(rv, acc.at[dv], scv, add=True)

                    @pl.when(q >= 1)
                    def _():
                        pltpu.make_async_copy(rv2, acc.at[dv2], scv2).wait()

                    @pl.when(q + 2 < nloc)
                    def _():
                        fetch(q + 2, pv2, nv2, sv2, dv2)
                        pltpu.async_copy(cur_h.at[sv2], rv2, gv2)

            return 0

        lax.fori_loop(0, (nloc + 2) // 3, it, 0)
        for b in range(3):
            pv, nv, sv, dv, rv, gv, scv = P[b]
            last1 = jnp.logical_and(nloc >= 1, lax.rem(nloc - 1, 3) == b)

            @pl.when(last1)
            def _(rv=rv, dv=dv, scv=scv):
                pltpu.make_async_copy(rv, acc.at[dv], scv).wait()

        plsc.subcore_barrier()
        for q in range(7):
            pltpu.sync_copy(acc.at[pl.ds(s * RH + q * 224, 224)],
                            r0.at[pl.ds(0, 224)])
            pltpu.sync_copy(r0.at[pl.ds(0, 224)],
                            out.at[pl.ds(c * HALF + s * RH + q * 224, 224)])

    return body(cur, pk, nm, c16)


def _sc_hop(curA, curB, pk, nm, c16):
    return (_sc_hop_half(curA, pk, nm, c16, DWA),
            _sc_hop_half(curB, pk, nm, c16, DWB))


def _elu(v):
    return jnp.where(v > 0, v, jnp.exp(jnp.minimum(v, 0.0)) - 1.0)


def _full2d(a):
    return pl.BlockSpec(a.shape, lambda i: (0,) * a.ndim)


def _tc_stage1(xp, idr, e0, e1, e2, d0, d1, p):
    f32 = jnp.float32

    def body(x_r, id_r, e0_r, e1_r, e2_r, d0_r, d1_r,
             wid_r, bid_r, w0_r, b0_r, we_r, be_r, g0_r, gb_r,
             ha_o, hb_o, dinv_o):
        idv = _elu(jnp.dot(id_r[...], wid_r[...],
                           preferred_element_type=f32) + bid_r[...])
        h0 = _elu(jnp.dot(x_r[...], w0_r[...],
                          preferred_element_type=f32) + b0_r[...])
        ecat = jnp.concatenate(
            [e0_r[...][:, :8], e1_r[...][:, :8], e2_r[...][:, :8]], axis=1)
        ev = _elu(jnp.dot(ecat, we_r[...],
                          preferred_element_type=f32) + be_r[...])
        hcat = jnp.concatenate([idv, h0, ev], axis=1)
        mu = jnp.mean(hcat, axis=1, keepdims=True)
        var = jnp.mean((hcat - mu) * (hcat - mu), axis=1, keepdims=True)
        hn = (hcat - mu) * lax.rsqrt(var + 1e-5) * g0_r[...] + gb_r[...]
        hp = jnp.concatenate([hn, jnp.zeros((256, DP - DREAL), f32)], axis=1)
        ha_o[...] = hp[:, :DWA]
        hb_o[...] = hp[:, DWA:]
        deg = d0_r[...] + d1_r[...]
        dinv_o[...] = jnp.where(deg > 0, lax.rsqrt(jnp.maximum(deg, 1e-30)),
                                0.0)

    wid = p['W_id']; bid = p['b_id'].reshape(1, -1)
    w0 = p['W0']; b0 = p['b0'].reshape(1, -1)
    we = p['W_emb']; be = p['b_emb'].reshape(1, -1)
    g0 = p['ln0_g'].reshape(1, -1); gb = p['ln0_b'].reshape(1, -1)
    row = lambda shp: pl.BlockSpec(shp, lambda i: (i, 0))
    row3 = pl.BlockSpec((1, 1, 256), lambda i: (i, 0, 0))
    return pl.pallas_call(
        body,
        grid=(GBLK,),
        in_specs=[row((256, 16)), row((256, 16)), row((256, 16)),
                  row((256, 16)), row((256, 16)), row3, row3,
                  _full2d(wid), _full2d(bid), _full2d(w0), _full2d(b0),
                  _full2d(we), _full2d(be), _full2d(g0), _full2d(gb)],
        out_specs=[row((256, DWA)), row((256, DWB)), row3],
        out_shape=[jax.ShapeDtypeStruct((NPAD, DWA), f32),
                   jax.ShapeDtypeStruct((NPAD, DWB), f32),
                   jax.ShapeDtypeStruct((GBLK, 1, 256), f32)],
    )(xp, idr, e0, e1, e2, d0, d1, wid, bid, w0, b0, we, be, g0, gb)


def _tc_final(halves, wall, p):
    f32 = jnp.float32

    def body(*refs):
        (ha_r, hb_r, c1a_r, c1b_r, c2a_r, c2b_r, c3a_r, c3b_r,
         wall_r, tb_r, g1_r, gb_r, w1_r, b1_r, o) = refs
        wr = wall_r[...]
        refs8 = (ha_r, hb_r, c1a_r, c1b_r, c2a_r, c2b_r, c3a_r, c3b_r)
        off = 0
        out = None
        for q, rr in enumerate(refs8):
            w = DWA if q % 2 == 0 else DWB
            term = jnp.dot(rr[...], wr[off:off + w, :],
                           preferred_element_type=f32)
            out = term if out is None else out + term
            off += w
        out = jnp.maximum(out + tb_r[...], 0.0)
        mu = jnp.mean(out, axis=1, keepdims=True)
        var = jnp.mean((out - mu) * (out - mu), axis=1, keepdims=True)
        out = (out - mu) * lax.rsqrt(var + 1e-5) * g1_r[...] + gb_r[...]
        y = jnp.dot(out, w1_r[...], preferred_element_type=f32) + b1_r[...]
        m = jnp.max(y, axis=1, keepdims=True)
        z = y - m
        o[...] = z - jnp.log(jnp.sum(jnp.exp(z), axis=1, keepdims=True))

    tb = p['tag_b'].reshape(1, -1)
    g1 = p['ln1_g'].reshape(1, -1); gb = p['ln1_b'].reshape(1, -1)
    w1 = p['W1']; b1 = p['b1'].reshape(1, -1)
    row = lambda shp: pl.BlockSpec(shp, lambda i: (i, 0))
    return pl.pallas_call(
        body,
        grid=(GBLK,),
        in_specs=[row((256, DWA)), row((256, DWB))] * 4 + [
            _full2d(wall), _full2d(tb),
            _full2d(g1), _full2d(gb), _full2d(w1), _full2d(b1)],
        out_specs=row((256, 2)),
        out_shape=jax.ShapeDtypeStruct((NPAD, 2), f32),
    )(*halves, wall, tb, g1, gb, w1, b1)


def kernel(x, edge_index, edge_weight, categories_value, params):
    p = params
    xp = jnp.pad(x, ((0, NPAD - N), (0, 0)))
    src = jnp.pad(edge_index[0], (0, EPAD - E))
    dst = jnp.pad(edge_index[1], (0, EPAD - E))
    ew = jnp.pad(edge_weight, (0, EPAD - E))
    cats = jnp.pad(categories_value.T.astype(jnp.int32),
                   ((0, 0), (0, NPAD - N))).reshape(4 * NPAD)
    embp = jnp.pad(p['emb_tables'], ((0, 0), (0, 0), (0, 8)))
    wall = jnp.pad(p['tag_W'], ((0, 0), (0, DP - DREAL), (0, 0)))
    wall = wall.reshape(4 * DP, -1)

    idr, e0r, e1r, e2r = _sc_gather(cats, p['id_table'], embp[0], embp[1],
                                    embp[2])
    deg2, cnts = _sc_degree(dst, ew)
    deg2 = deg2.reshape(2, NPAD)
    ha, hb, dinv3 = _tc_stage1(xp, idr, e0r, e1r, e2r,
                               deg2[0].reshape(GBLK, 1, 256),
                               deg2[1].reshape(GBLK, 1, 256), p)
    dinv = dinv3.reshape(NPAD)
    nrm = _sc_norm(dinv, src, dst, ew)
    pk, nm, c16 = _sc_part(src, dst, nrm, cnts)
    c1a, c1b = _sc_hop(ha, hb, pk, nm, c16)
    c2a, c2b = _sc_hop(c1a, c1b, pk, nm, c16)
    c3a, c3b = _sc_hop(c2a, c2b, pk, nm, c16)
    out = _tc_final((ha, hb, c1a, c1b, c2a, c2b, c3a, c3b), wall, p)
    return out[:N]


# speedup vs baseline: 13.0575x; 1.0005x over previous
import functools

import jax
import jax.numpy as jnp
from jax import lax
from jax.experimental import pallas as pl
from jax.experimental.pallas import tpu as pltpu
from jax.experimental.pallas import tpu_sc as plsc

N = 50000
NPAD = 50176
E = 800000
EPAD = 802816
DP = 80
DWA = 48
DWB = 32
DREAL = 72
HALF = NPAD // 2
RT = NPAD // 32
RH = HALF // 16
ET32 = EPAD // 32
ECH = 3136
ET16 = EPAD // 16
HCHA = 320
HCHB = 512
GBLK = NPAD // 256
PKSH = EPAD + 256


def _mesh():
    return plsc.VectorSubcoreMesh(core_axis_name="c", subcore_axis_name="s",
                                  num_cores=2, num_subcores=16)


_SCPARAMS = dict(
    compiler_params=pltpu.CompilerParams(use_tc_tiling_on_sc=False,
                                         needs_layout_passes=False))


def _zero16():
    return jnp.zeros((16,), jnp.float32)


def _iota16():
    return lax.iota(jnp.int32, 16)


def _sc_gather(cats, id_table, emb0, emb1, emb2):
    f32 = jnp.float32
    out_t = [jax.ShapeDtypeStruct((NPAD, 16), f32) for _ in range(4)]

    @functools.partial(
        pl.kernel,
        out_type=out_t,
        mesh=_mesh(),
        scratch_types=[
            pltpu.VMEM((RT,), jnp.int32),
            pltpu.VMEM((RT, 16), f32),
            pltpu.SemaphoreType.DMA,
        ],
        **_SCPARAMS,
    )
    def body(cats_h, t0, t1, t2, t3, o0, o1, o2, o3, idx_v, rows_v, sem):
        c = lax.axis_index("c")
        s = lax.axis_index("s")
        base = (s * 2 + c) * RT
        for k, tbl, out in ((0, t0, o0), (1, t1, o1), (2, t2, o2), (3, t3, o3)):
            pltpu.sync_copy(cats_h.at[pl.ds(k * NPAD + base, RT)], idx_v)
            pltpu.async_copy(tbl.at[idx_v], rows_v, sem).wait()
            pltpu.sync_copy(rows_v, out.at[pl.ds(base, RT)])

    return body(cats, id_table, emb0, emb1, emb2)


def _sc_degree(dst, ew):
    f32 = jnp.float32
    i32 = jnp.int32
    out_t = [jax.ShapeDtypeStruct((2 * NPAD,), f32),
             jax.ShapeDtypeStruct((512,), i32)]

    @functools.partial(
        pl.kernel,
        out_type=out_t,
        mesh=_mesh(),
        scratch_types=[
            pltpu.VMEM((NPAD,), f32),
            pltpu.VMEM((ECH,), i32),
            pltpu.VMEM((ECH,), f32),
            pltpu.VMEM((ECH,), f32),
            pltpu.VMEM((16,), i32),
            pltpu.VMEM_SHARED((16 * NPAD,), f32),
        ],
        **_SCPARAMS,
    )
    def body(dst_h, ew_h, out, cnt_o, acc_v, idx_v, val_v, z_v, cnt_v, part):
        c = lax.axis_index("c")
        s = lax.axis_index("s")
        wid = s * 2 + c

        def za(i, _):
            acc_v[pl.ds(i * 16, 16)] = _zero16()
            return 0

        lax.fori_loop(0, NPAD // 16, za, 0)

        def chunk(j, cnt):
            base = wid * ET32 + j * ECH
            pltpu.sync_copy(dst_h.at[pl.ds(base, ECH)], idx_v)
            pltpu.sync_copy(ew_h.at[pl.ds(base, ECH)], val_v)

            def grp(g, cn):
                sl = pl.ds(g * 16, 16)
                di = idx_v[sl]
                plsc.addupdate_scatter(acc_v, [di], val_v[sl])
                return cn + jnp.where(di < HALF, 1, 0).astype(i32)

            return lax.fori_loop(0, ECH // 16, grp, cnt)

        cnt = lax.fori_loop(0, ET32 // ECH, chunk, jnp.zeros((16,), i32))
        cnt_v[...] = cnt
        pltpu.sync_copy(cnt_v, cnt_o.at[pl.ds(wid * 16, 16)])
        pltpu.sync_copy(acc_v, part.at[pl.ds(s * NPAD, NPAD)])
        plsc.subcore_barrier()

        def red(r, _):
            pltpu.sync_copy(part.at[pl.ds(r * NPAD + s * ECH, ECH)], val_v)

            def add(g, _):
                sl = pl.ds(g * 16, 16)
                z_v[sl] = z_v[sl] + val_v[sl]
                return 0

            lax.fori_loop(0, ECH // 16, add, 0)
            return 0

        def zb(i, _):
            z_v[pl.ds(i * 16, 16)] = _zero16()
            return 0

        lax.fori_loop(0, ECH // 16, zb, 0)
        lax.fori_loop(0, 16, red, 0)
        pltpu.sync_copy(z_v, out.at[pl.ds(c * NPAD + s * ECH, ECH)])

    return body(dst, ew)


def _sc_norm(dinv, src, dst, ew):
    f32 = jnp.float32
    i32 = jnp.int32

    @functools.partial(
        pl.kernel,
        out_type=jax.ShapeDtypeStruct((EPAD,), f32),
        mesh=_mesh(),
        scratch_types=[
            pltpu.VMEM((NPAD,), f32),
            pltpu.VMEM((ECH,), i32),
            pltpu.VMEM((ECH,), i32),
            pltpu.VMEM((ECH,), f32),
            pltpu.VMEM((ECH,), f32),
            pltpu.VMEM_SHARED((NPAD,), f32),
        ],
        **_SCPARAMS,
    )
    def body(dinv_h, src_h, dst_h, ew_h, norm_o,
             dinv_v, src_v, dst_v, ew_v, nrm_v, dinv_sh):
        c = lax.axis_index("c")
        s = lax.axis_index("s")
        wid = s * 2 + c
        pltpu.sync_copy(dinv_h.at[pl.ds(s * ECH, ECH)],
                        dinv_v.at[pl.ds(s * ECH, ECH)])
        pltpu.sync_copy(dinv_v.at[pl.ds(s * ECH, ECH)],
                        dinv_sh.at[pl.ds(s * ECH, ECH)])
        plsc.subcore_barrier()
        pltpu.sync_copy(dinv_sh, dinv_v)

        def chunk(j, _):
            base = wid * ET32 + j * ECH
            pltpu.sync_copy(src_h.at[pl.ds(base, ECH)], src_v)
            pltpu.sync_copy(dst_h.at[pl.ds(base, ECH)], dst_v)
            pltpu.sync_copy(ew_h.at[pl.ds(base, ECH)], ew_v)

            def grp(g, _):
                sl = pl.ds(g * 16, 16)
                a = plsc.load_gather(dinv_v, [src_v[sl]])
                b = plsc.load_gather(dinv_v, [dst_v[sl]])
                nrm_v[sl] = a * ew_v[sl] * b
                return 0

            lax.fori_loop(0, ECH // 16, grp, 0)
            pltpu.sync_copy(nrm_v, norm_o.at[pl.ds(base, ECH)])
            return 0

        lax.fori_loop(0, ET32 // ECH, chunk, 0)

    return body(dinv, src, dst, ew)


def _sc_part(src, dst, nrm, cnts):
    f32 = jnp.float32
    i32 = jnp.int32
    out_t = [jax.ShapeDtypeStruct((2 * EPAD,), i32),
             jax.ShapeDtypeStruct((2 * EPAD,), f32),
             jax.ShapeDtypeStruct((16,), i32)]

    @functools.partial(
        pl.kernel,
        out_type=out_t,
        mesh=_mesh(),
        scratch_types=[
            pltpu.VMEM((512,), i32),
            pltpu.VMEM((ECH,), i32),
            pltpu.VMEM((ECH,), i32),
            pltpu.VMEM((ECH,), f32),
            pltpu.VMEM((ECH,), i32),
            pltpu.VMEM((ECH,), i32),
            pltpu.VMEM_SHARED((PKSH,), i32),
            pltpu.VMEM_SHARED((PKSH,), f32),
        ],
        **_SCPARAMS,
    )
    def body(src_h, dst_h, nrm_h, cnts_h, pk_o, nm_o, c16_o,
             cv, src_v, dst_v, nrm_v, pk_v, pos_v, pk_sh, nm_sh):
        c = lax.axis_index("c")
        s = lax.axis_index("s")
        pltpu.sync_copy(cnts_h, cv)

        def pw(w, carry):
            tot, pre = carry
            rs = lax.reduce_sum(cv[pl.ds(w * 16, 16)], axes=(0,))
            pre = pre + jnp.where(w < 2 * s, rs, 0)
            return tot + rs, pre

        c0_total, s_pre = lax.fori_loop(0, 32, pw,
                                        (jnp.int32(0), jnp.int32(0)))
        my_off = jnp.where(c == 0, s_pre, 2 * s * ET32 - s_pre)
        dummy = EPAD + s * 16
        zslice = EPAD // 16

        def zb(i, _):
            pk_v[pl.ds(i * 16, 16)] = jnp.zeros((16,), i32)
            nrm_v[pl.ds(i * 16, 16)] = _zero16()
            return 0

        lax.fori_loop(0, ECH // 16, zb, 0)
        for q in range(zslice // ECH):
            pltpu.sync_copy(pk_v, pk_sh.at[pl.ds(s * zslice + q * ECH, ECH)])
            pltpu.sync_copy(nrm_v, nm_sh.at[pl.ds(s * zslice + q * ECH, ECH)])
        pltpu.sync_copy(pk_v.at[pl.ds(0, 16)], pk_sh.at[pl.ds(dummy, 16)])
        pltpu.sync_copy(nrm_v.at[pl.ds(0, 16)], nm_sh.at[pl.ds(dummy, 16)])
        plsc.subcore_barrier()

        def chunk(j, p):
            base = s * ET16 + j * ECH
            pltpu.sync_copy(src_h.at[pl.ds(base, ECH)], src_v)
            pltpu.sync_copy(dst_h.at[pl.ds(base, ECH)], dst_v)
            pltpu.sync_copy(nrm_h.at[pl.ds(base, ECH)], nrm_v)

            def grp(g, pp):
                sl = pl.ds(g * 16, 16)
                si = src_v[sl]
                di = dst_v[sl]
                m0 = di < HALF
                m = jnp.where(c == 0, m0, jnp.logical_not(m0))
                dl = jnp.where(m0, di, di - HALF)
                pk_v[sl] = jnp.bitwise_or(jnp.left_shift(si, 15), dl)
                ones = jnp.where(m, 1, 0).astype(i32)
                incl = plsc.cumsum(ones)
                pos_v[sl] = jnp.where(m, my_off + pp + incl - 1, dummy)
                return pp + lax.reduce_sum(ones, axes=(0,))

            p = lax.fori_loop(0, ECH // 16, grp, p)
            pltpu.sync_copy(pk_v, pk_sh.at[pos_v])
            pltpu.sync_copy(nrm_v, nm_sh.at[pos_v])
            return p

        lax.fori_loop(0, ET16 // ECH, chunk, jnp.int32(0))
        plsc.subcore_barrier()

        def ocp(q, _):
            o = s * zslice + q * ECH
            pltpu.sync_copy(pk_sh.at[pl.ds(o, ECH)], pk_v)
            pltpu.sync_copy(pk_v, pk_o.at[pl.ds(c * EPAD + o, ECH)])
            pltpu.sync_copy(nm_sh.at[pl.ds(o, ECH)], nrm_v)
            pltpu.sync_copy(nrm_v, nm_o.at[pl.ds(c * EPAD + o, ECH)])
            return 0

        lax.fori_loop(0, zslice // ECH, ocp, 0)

        @pl.when(jnp.logical_and(c == 0, s == 0))
        def _():
            it = _iota16()
            val = jnp.where(it == 0, c0_total,
                            jnp.where(it == 1, EPAD - c0_total, 0))
            pk_v[pl.ds(0, 16)] = val
            pltpu.sync_copy(pk_v.at[pl.ds(0, 16)], c16_o)

    return body(src, dst, nrm, cnts)


def _sc_hop_half(cur, pk, nm, c16, dw):
    f32 = jnp.float32
    i32 = jnp.int32
    HCH = HCHA if dw == DWA else HCHB

    @functools.partial(
        pl.kernel,
        out_type=jax.ShapeDtypeStruct((NPAD, dw), f32),
        mesh=_mesh(),
        scratch_types=[
            pltpu.VMEM((16,), i32),
            pltpu.VMEM((HCH,), i32), pltpu.VMEM((HCH,), f32),
            pltpu.VMEM((HCH,), i32), pltpu.VMEM((HCH,), i32),
            pltpu.VMEM((HCH,), i32), pltpu.VMEM((HCH,), f32),
            pltpu.VMEM((HCH,), i32), pltpu.VMEM((HCH,), i32),
            pltpu.VMEM((HCH,), i32), pltpu.VMEM((HCH,), f32),
            pltpu.VMEM((HCH,), i32), pltpu.VMEM((HCH,), i32),
            pltpu.VMEM((HCH, dw), f32), pltpu.VMEM((HCH, dw), f32),
            pltpu.VMEM((HCH, dw), f32),
            pltpu.VMEM_SHARED((HALF, dw), f32),
            pltpu.SemaphoreType.DMA, pltpu.SemaphoreType.DMA,
            pltpu.SemaphoreType.DMA, pltpu.SemaphoreType.DMA,
            pltpu.SemaphoreType.DMA, pltpu.SemaphoreType.DMA,
        ],
        **_SCPARAMS,
    )
    def body(cur_h, pk_h, nm_h, c16_h, out, cv,
             p0, n0, s0, d0, p1, n1, s1, d1, p2, n2, s2, d2,
             r0, r1, r2, acc, g0, g1, g2, c0, c1, c2):
        c = lax.axis_index("c")
        s = lax.axis_index("s")
        pltpu.sync_copy(c16_h, cv)
        cnt = lax.reduce_max(jnp.where(_iota16() == c, cv[...], 0), axes=(0,))
        total = (cnt + HCH - 1) // HCH
        nloc = (total - s + 15) // 16
        P = ((p0, n0, s0, d0, r0, g0, c0),
             (p1, n1, s1, d1, r1, g1, c1),
             (p2, n2, s2, d2, r2, g2, c2))

        def zr(i, _):
            for j in range(dw // 16):
                r0[i, pl.ds(j * 16, 16)] = _zero16()
            return 0

        lax.fori_loop(0, HCH, zr, 0)
        for q in range(7):
            pltpu.sync_copy(r0.at[pl.ds(0, 224)],
                            acc.at[pl.ds(s * RH + q * 224, 224)])
        plsc.subcore_barrier()

        def fetch(i, pv, nv, sv, dv):
            b = c * EPAD + (s + i * 16) * HCH
            pltpu.sync_copy(pk_h.at[pl.ds(b, HCH)], pv)
            pltpu.sync_copy(nm_h.at[pl.ds(b, HCH)], nv)

            def unp(g, _):
                sl = pl.ds(g * 16, 16)
                w = pv[sl]
                sv[sl] = jnp.right_shift(w, 15)
                dv[sl] = jnp.bitwise_and(w, 32767)
                return 0

            lax.fori_loop(0, HCH // 16, unp, 0)

        def scale(rv, nv):
            def grp(g, _):
                nvv = nv[pl.ds(g * 16, 16)]
                for i in range(16):
                    e = g * 16 + i
                    bb = nvv.at[jnp.full((16,), i, i32)].get(
                        mode="promise_in_bounds")
                    for jj in range(dw // 16):
                        sl = pl.ds(jj * 16, 16)
                        rv[e, sl] = rv[e, sl] * bb
                return 0

            lax.fori_loop(0, HCH // 16, grp, 0)

        for b in range(2):
            pv, nv, sv, dv, rv, gv, _cv = P[b]

            @pl.when(nloc > b)
            def _(b=b, pv=pv, nv=nv, sv=sv, dv=dv, rv=rv, gv=gv):
                fetch(b, pv, nv, sv, dv)
                pltpu.async_copy(cur_h.at[sv], rv, gv)

        def it(i, _):
            for b in range(3):
                q = i * 3 + b
                pv, nv, sv, dv, rv, gv, scv = P[b]
                b2 = (b + 2) % 3
                pv2, nv2, sv2, dv2, rv2, gv2, scv2 = P[b2]

                @pl.when(q < nloc)
                def _(q=q, pv=pv, nv=nv, sv=sv, dv=dv, rv=rv, gv=gv,
                      scv=scv, pv2=pv2, nv2=nv2, sv2=sv2, dv2=dv2,
                      rv2=rv2, gv2=gv2, scv2=scv2):
                    pltpu.make_async_copy(cur_h.at[sv], rv, gv).wait()
                    scale(rv, nv)
                    pltpu.async_copy(rv, acc.at[dv], scv, add=True)

                    @pl.when(q >= 1)
                    def _():
                        pltpu.make_async_copy(rv2, acc.at[dv2], scv2).wait()

                    @pl.when(q + 2 < nloc)
                    def _():
                        fetch(q + 2, pv2, nv2, sv2, dv2)
                        pltpu.async_copy(cur_h.at[sv2], rv2, gv2)

            return 0

        lax.fori_loop(0, (nloc + 2) // 3, it, 0)
        for b in range(3):
            pv, nv, sv, dv, rv, gv, scv = P[b]
            last1 = jnp.logical_and(nloc >= 1, lax.rem(nloc - 1, 3) == b)

            @pl.when(last1)
            def _(rv=rv, dv=dv, scv=scv):
                pltpu.make_async_copy(rv, acc.at[dv], scv).wait()

        plsc.subcore_barrier()
        for q in range(7):
            pltpu.sync_copy(acc.at[pl.ds(s * RH + q * 224, 224)],
                            r0.at[pl.ds(0, 224)])
            pltpu.sync_copy(r0.at[pl.ds(0, 224)],
                            out.at[pl.ds(c * HALF + s * RH + q * 224, 224)])

    return body(cur, pk, nm, c16)


def _sc_hop(curA, curB, pk, nm, c16):
    return (_sc_hop_half(curA, pk, nm, c16, DWA),
            _sc_hop_half(curB, pk, nm, c16, DWB))


def _elu(v):
    return jnp.where(v > 0, v, jnp.exp(jnp.minimum(v, 0.0)) - 1.0)


def _full2d(a):
    return pl.BlockSpec(a.shape, lambda i: (0,) * a.ndim)


def _tc_stage1(xp, idr, e0, e1, e2, d0, d1, p):
    f32 = jnp.float32

    def body(x_r, id_r, e0_r, e1_r, e2_r, d0_r, d1_r,
             wid_r, bid_r, w0_r, b0_r, we_r, be_r, g0_r, gb_r,
             ha_o, hb_o, dinv_o):
        idv = _elu(jnp.dot(id_r[...], wid_r[...],
                           preferred_element_type=f32) + bid_r[...])
        h0 = _elu(jnp.dot(x_r[...], w0_r[...],
                          preferred_element_type=f32) + b0_r[...])
        ecat = jnp.concatenate(
            [e0_r[...][:, :8], e1_r[...][:, :8], e2_r[...][:, :8]], axis=1)
        ev = _elu(jnp.dot(ecat, we_r[...],
                          preferred_element_type=f32) + be_r[...])
        hcat = jnp.concatenate([idv, h0, ev], axis=1)
        mu = jnp.mean(hcat, axis=1, keepdims=True)
        var = jnp.mean((hcat - mu) * (hcat - mu), axis=1, keepdims=True)
        hn = (hcat - mu) * lax.rsqrt(var + 1e-5) * g0_r[...] + gb_r[...]
        hp = jnp.concatenate([hn, jnp.zeros((256, DP - DREAL), f32)], axis=1)
        ha_o[...] = hp[:, :DWA]
        hb_o[...] = hp[:, DWA:]
        deg = d0_r[...] + d1_r[...]
        dinv_o[...] = jnp.where(deg > 0, lax.rsqrt(jnp.maximum(deg, 1e-30)),
                                0.0)

    wid = p['W_id']; bid = p['b_id'].reshape(1, -1)
    w0 = p['W0']; b0 = p['b0'].reshape(1, -1)
    we = p['W_emb']; be = p['b_emb'].reshape(1, -1)
    g0 = p['ln0_g'].reshape(1, -1); gb = p['ln0_b'].reshape(1, -1)
    row = lambda shp: pl.BlockSpec(shp, lambda i: (i, 0))
    row3 = pl.BlockSpec((1, 1, 256), lambda i: (i, 0, 0))
    return pl.pallas_call(
        body,
        grid=(GBLK,),
        in_specs=[row((256, 16)), row((256, 16)), row((256, 16)),
                  row((256, 16)), row((256, 16)), row3, row3,
                  _full2d(wid), _full2d(bid), _full2d(w0), _full2d(b0),
                  _full2d(we), _full2d(be), _full2d(g0), _full2d(gb)],
        out_specs=[row((256, DWA)), row((256, DWB)), row3],
        out_shape=[jax.ShapeDtypeStruct((NPAD, DWA), f32),
                   jax.ShapeDtypeStruct((NPAD, DWB), f32),
                   jax.ShapeDtypeStruct((GBLK, 1, 256), f32)],
    )(xp, idr, e0, e1, e2, d0, d1, wid, bid, w0, b0, we, be, g0, gb)


def _tc_final(halves, wall, p):
    f32 = jnp.float32

    def body(*refs):
        (ha_r, hb_r, c1a_r, c1b_r, c2a_r, c2b_r, c3a_r, c3b_r,
         wall_r, tb_r, g1_r, gb_r, w1_r, b1_r, o) = refs
        wr = wall_r[...]
        refs8 = (ha_r, hb_r, c1a_r, c1b_r, c2a_r, c2b_r, c3a_r, c3b_r)
        off = 0
        out = None
        for q, rr in enumerate(refs8):
            w = DWA if q % 2 == 0 else DWB
            term = jnp.dot(rr[...], wr[off:off + w, :],
                           preferred_element_type=f32)
            out = term if out is None else out + term
            off += w
        out = jnp.maximum(out + tb_r[...], 0.0)
        mu = jnp.mean(out, axis=1, keepdims=True)
        var = jnp.mean((out - mu) * (out - mu), axis=1, keepdims=True)
        out = (out - mu) * lax.rsqrt(var + 1e-5) * g1_r[...] + gb_r[...]
        y = jnp.dot(out, w1_r[...], preferred_element_type=f32) + b1_r[...]
        m = jnp.max(y, axis=1, keepdims=True)
        z = y - m
        o[...] = z - jnp.log(jnp.sum(jnp.exp(z), axis=1, keepdims=True))

    tb = p['tag_b'].reshape(1, -1)
    g1 = p['ln1_g'].reshape(1, -1); gb = p['ln1_b'].reshape(1, -1)
    w1 = p['W1']; b1 = p['b1'].reshape(1, -1)
    row = lambda shp: pl.BlockSpec(shp, lambda i: (i, 0))
    return pl.pallas_call(
        body,
        grid=(GBLK,),
        in_specs=[row((256, DWA)), row((256, DWB))] * 4 + [
            _full2d(wall), _full2d(tb),
            _full2d(g1), _full2d(gb), _full2d(w1), _full2d(b1)],
        out_specs=row((256, 2)),
        out_shape=jax.ShapeDtypeStruct((NPAD, 2), f32),
    )(*halves, wall, tb, g1, gb, w1, b1)


def kernel(x, edge_index, edge_weight, categories_value, params):
    p = params
    xp = jnp.pad(x, ((0, NPAD - N), (0, 0)))
    src = jnp.pad(edge_index[0], (0, EPAD - E))
    dst = jnp.pad(edge_index[1], (0, EPAD - E))
    ew = jnp.pad(edge_weight, (0, EPAD - E))
    cats = jnp.pad(categories_value.T.astype(jnp.int32),
                   ((0, 0), (0, NPAD - N))).reshape(4 * NPAD)
    embp = jnp.pad(p['emb_tables'], ((0, 0), (0, 0), (0, 8)))
    wall = jnp.pad(p['tag_W'], ((0, 0), (0, DP - DREAL), (0, 0)))
    wall = wall.reshape(4 * DP, -1)

    idr, e0r, e1r, e2r = _sc_gather(cats, p['id_table'], embp[0], embp[1],
                                    embp[2])
    deg2, cnts = _sc_degree(dst, ew)
    deg2 = deg2.reshape(2, NPAD)
    ha, hb, dinv3 = _tc_stage1(xp, idr, e0r, e1r, e2r,
                               deg2[0].reshape(GBLK, 1, 256),
                               deg2[1].reshape(GBLK, 1, 256), p)
    dinv = dinv3.reshape(NPAD)
    nrm = _sc_norm(dinv, src, dst, ew)
    pk, nm, c16 = _sc_part(src, dst, nrm, cnts)
    c1a, c1b = _sc_hop(ha, hb, pk, nm, c16)
    c2a, c2b = _sc_hop(c1a, c1b, pk, nm, c16)
    c3a, c3b = _sc_hop(c2a, c2b, pk, nm, c16)
    out = _tc_final((ha, hb, c1a, c1b, c2a, c2b, c3a, c3b), wall, p)
    return out[:N]
